# Initial kernel scaffold; baseline (speedup 1.0000x reference)
import jax
import jax.numpy as jnp
from jax.experimental import pallas as pl


def kernel(x, ec1_w, ec1_g, ec1_b, ec2_mw, ec2_g, ec2_b, ec2_s1w, ec2_s1b, ec2_s2w, ec2_s2b, ec3_mw, ec3_g, ec3_b, ec3_s1w, ec3_s1b, ec3_s2w, ec3_s2b, att_w1, att_b1, att_w2, att_b2, fc_w, fc_b):
    raise NotImplementedError("write your pallas kernel here")


# speedup vs baseline: 4.6080x; 4.6080x over previous
import functools

import jax
import jax.numpy as jnp
from jax import lax
from jax.experimental import pallas as pl
from jax.experimental.pallas import tpu as pltpu
from jax.experimental.pallas import tpu_sc as plsc

_K = 10
_TAU = 0.2
_EPS = 1e-5


def _knn_uv1_body(x_ref, wg_ref, wc_ref, idx_ref, u_ref, v_ref):
    b = pl.program_id(0)
    N = x_ref.shape[1]
    t = x_ref[0]
    lane = lax.broadcasted_iota(jnp.int32, t.shape, 1)
    c = jnp.where(lane < 3, t, 0.0)
    sq = jnp.sum(c * c, axis=1, keepdims=True)
    dot = lax.dot_general(c, c, (((1,), (1,)), ((), ())),
                          preferred_element_type=jnp.float32)
    v = sq - 2.0 * dot
    row = lax.broadcasted_iota(jnp.int32, (N, N), 0)
    base = b * N
    for j in range(_K + 1):
        mn = jnp.min(v, axis=0, keepdims=True)
        am = jnp.min(jnp.where(v <= mn, row, N), axis=0, keepdims=True)
        idx_ref[0, j:j + 1, :] = am + base
        v = jnp.where(row == am, jnp.float32(1e30), v)
    idx_ref[0, _K + 1:16, :] = jnp.full((16 - _K - 1, N), base, jnp.int32)
    u_ref[0] = jnp.dot(t, wg_ref[...], preferred_element_type=jnp.float32)
    v_ref[0] = jnp.dot(t, wc_ref[...], preferred_element_type=jnp.float32)


def _knn_uv1(x, wg1t, wc1t):
    B, N, _ = x.shape
    Cu, Cv = wg1t.shape[1], wc1t.shape[1]
    return pl.pallas_call(
        _knn_uv1_body,
        grid=(B,),
        in_specs=[
            pl.BlockSpec((1, N, 5), lambda b: (b, 0, 0)),
            pl.BlockSpec((5, Cu), lambda b: (0, 0)),
            pl.BlockSpec((5, Cv), lambda b: (0, 0)),
        ],
        out_specs=[
            pl.BlockSpec((1, 16, N), lambda b: (b, 0, 0)),
            pl.BlockSpec((1, N, Cu), lambda b: (b, 0, 0)),
            pl.BlockSpec((1, N, Cv), lambda b: (b, 0, 0)),
        ],
        out_shape=[
            jax.ShapeDtypeStruct((B, 16, N), jnp.int32),
            jax.ShapeDtypeStruct((B, N, Cu), jnp.float32),
            jax.ShapeDtypeStruct((B, N, Cv), jnp.float32),
        ],
    )(x, wg1t, wc1t)


def _uv_body(t_ref, wgh_ref, wch_ref, wgs_ref, wcs_ref, sb_ref,
             uh_ref, vh_ref, us_ref, vs_ref):
    t = t_ref[0]
    uh_ref[0] = jnp.dot(t, wgh_ref[...], preferred_element_type=jnp.float32)
    vh_ref[0] = jnp.dot(t, wch_ref[...], preferred_element_type=jnp.float32)
    us_ref[0] = jnp.dot(t, wgs_ref[...], preferred_element_type=jnp.float32)
    vs_ref[0] = (jnp.dot(t, wcs_ref[...], preferred_element_type=jnp.float32)
                 + sb_ref[...])


def _uv(t, wght, wcht, wgst, wcst, s1b):
    B, N, Cin = t.shape
    Ch, Cs = wght.shape[1], wgst.shape[1]
    return pl.pallas_call(
        _uv_body,
        grid=(B,),
        in_specs=[
            pl.BlockSpec((1, N, Cin), lambda b: (b, 0, 0)),
            pl.BlockSpec((Cin, Ch), lambda b: (0, 0)),
            pl.BlockSpec((Cin, Ch), lambda b: (0, 0)),
            pl.BlockSpec((Cin, Cs), lambda b: (0, 0)),
            pl.BlockSpec((Cin, Cs), lambda b: (0, 0)),
            pl.BlockSpec((1, Cs), lambda b: (0, 0)),
        ],
        out_specs=[
            pl.BlockSpec((1, N, Ch), lambda b: (b, 0, 0)),
            pl.BlockSpec((1, N, Ch), lambda b: (b, 0, 0)),
            pl.BlockSpec((1, N, Cs), lambda b: (b, 0, 0)),
            pl.BlockSpec((1, N, Cs), lambda b: (b, 0, 0)),
        ],
        out_shape=[
            jax.ShapeDtypeStruct((B, N, Ch), jnp.float32),
            jax.ShapeDtypeStruct((B, N, Ch), jnp.float32),
            jax.ShapeDtypeStruct((B, N, Cs), jnp.float32),
            jax.ShapeDtypeStruct((B, N, Cs), jnp.float32),
        ],
    )(t, wght, wcht, wgst, wcst, s1b)


_CH = 128
_NW = 32


def _sc_gather2(idx, th, ts, j0):
    B, _, N = idx.shape
    Ch, Cs = th.shape[1], ts.shape[1]
    M = B * _K * N
    n_chunks = M // _CH
    per_w = n_chunks // _NW
    cpb = _K * N // _CH
    cpp = N // _CH
    mesh = plsc.VectorSubcoreMesh(core_axis_name="c", subcore_axis_name="s")

    @functools.partial(
        pl.kernel,
        mesh=mesh,
        out_type=(jax.ShapeDtypeStruct((M, Ch), jnp.float32),
                  jax.ShapeDtypeStruct((M, Cs), jnp.float32)),
        scratch_types=[
            pltpu.VMEM((_CH,), jnp.int32),
            pltpu.VMEM((_CH, Ch), jnp.float32),
            pltpu.VMEM((_CH, Cs), jnp.float32),
            pltpu.SemaphoreType.DMA,
            pltpu.SemaphoreType.DMA,
        ],
    )
    def k(idx_hbm, th_hbm, ts_hbm, oh_hbm, os_hbm, idx_v, rh_v, rs_v, sh, ss):
        wid = lax.axis_index("s") * 2 + lax.axis_index("c")
        for r in range(per_w):
            c = wid * per_w + r
            b = c // cpb
            rem = c % cpb
            j = j0 + rem // cpp
            n0 = (rem % cpp) * _CH
            pltpu.sync_copy(idx_hbm.at[b, j, pl.ds(n0, _CH)], idx_v)
            cph = pltpu.async_copy(th_hbm.at[idx_v], rh_v, sh)
            cps = pltpu.async_copy(ts_hbm.at[idx_v], rs_v, ss)
            cph.wait()
            cps.wait()
            pltpu.sync_copy(rh_v, oh_hbm.at[pl.ds(c * _CH, _CH), :])
            pltpu.sync_copy(rs_v, os_hbm.at[pl.ds(c * _CH, _CH), :])

    return k(idx, th, ts)


def _sc_gather1(idx, tb, j0):
    B, _, N = idx.shape
    C = tb.shape[1]
    M = B * _K * N
    n_chunks = M // _CH
    per_w = n_chunks // _NW
    cpb = _K * N // _CH
    cpp = N // _CH
    mesh = plsc.VectorSubcoreMesh(core_axis_name="c", subcore_axis_name="s")

    @functools.partial(
        pl.kernel,
        mesh=mesh,
        out_type=jax.ShapeDtypeStruct((M, C), jnp.float32),
        scratch_types=[
            pltpu.VMEM((_CH,), jnp.int32),
            pltpu.VMEM((_CH, C), jnp.float32),
            pltpu.SemaphoreType.DMA,
        ],
    )
    def k(idx_hbm, tb_hbm, o_hbm, idx_v, r_v, sem):
        wid = lax.axis_index("s") * 2 + lax.axis_index("c")
        for r in range(per_w):
            c = wid * per_w + r
            b = c // cpb
            rem = c % cpb
            j = j0 + rem // cpp
            n0 = (rem % cpp) * _CH
            pltpu.sync_copy(idx_hbm.at[b, j, pl.ds(n0, _CH)], idx_v)
            pltpu.async_copy(tb_hbm.at[idx_v], r_v, sem).wait()
            pltpu.sync_copy(r_v, o_hbm.at[pl.ds(c * _CH, _CH), :])

    return k(idx, tb)


def _ep1_body(e_ref, v_ref, g_ref, bt_ref, o_ref):
    C = v_ref.shape[-1]
    y = e_ref[...][..., 0:C] + v_ref[...][:, None]
    C = y.shape[-1]
    y2 = y.reshape(-1, C)
    minv = 1.0 / y2.shape[0]
    m = jnp.sum(y2, axis=0, keepdims=True) * minv
    var = jnp.sum(y2 * y2, axis=0, keepdims=True) * minv - m * m
    scale = g_ref[...] * lax.rsqrt(var + _EPS)
    sh = bt_ref[...] - m * scale
    h = jnp.maximum(y * scale[0] + sh[0], 0.0)
    o_ref[...] = jnp.max(h, axis=1)


def _ep1(e, v, g, bt):
    B, N, C = v.shape
    return pl.pallas_call(
        _ep1_body,
        out_shape=jax.ShapeDtypeStruct((B, N, C), jnp.float32),
    )(e, v, g, bt)


def _as_body(eh_ref, es_ref, vh_ref, vs_ref, w_ref, al_ref, sm_ref):
    nb = es_ref.shape[2]
    ys = es_ref[0] + vs_ref[0][None]
    l = jnp.sum(jnp.maximum(ys, 0.0) * w_ref[...][0], axis=-1) * (1.0 / _TAU)
    mx = jnp.max(l, axis=0, keepdims=True)
    ex = jnp.exp(l - mx)
    al = ex / jnp.sum(ex, axis=0, keepdims=True)
    al_ref[0, 0:_K, :] = al
    al_ref[0, _K:16, :] = jnp.zeros((16 - _K, nb), jnp.float32)
    yh = eh_ref[0] + vh_ref[0][None]
    Ch = yh.shape[-1]
    y2 = yh.reshape(-1, Ch)
    s1 = jnp.sum(y2, axis=0, keepdims=True)
    s2 = jnp.sum(y2 * y2, axis=0, keepdims=True)

    @pl.when(jnp.logical_and(pl.program_id(0) == 0, pl.program_id(1) == 0))
    def _():
        sm_ref[...] = jnp.zeros_like(sm_ref)

    sm_ref[0:1, :] += s1
    sm_ref[1:2, :] += s2


def _alpha_stats(eh, es, vh, vs, s2w, nb):
    B, _, N, Ch = eh.shape
    Cs = es.shape[-1]
    nch = N // nb
    return pl.pallas_call(
        _as_body,
        grid=(B, nch),
        in_specs=[
            pl.BlockSpec((1, _K, nb, Ch), lambda b, i: (b, 0, i, 0)),
            pl.BlockSpec((1, _K, nb, Cs), lambda b, i: (b, 0, i, 0)),
            pl.BlockSpec((1, nb, Ch), lambda b, i: (b, i, 0)),
            pl.BlockSpec((1, nb, Cs), lambda b, i: (b, i, 0)),
            pl.BlockSpec((1, Cs), lambda b, i: (0, 0)),
        ],
        out_specs=[
            pl.BlockSpec((1, 16, nb), lambda b, i: (b, 0, i)),
            pl.BlockSpec((8, Ch), lambda b, i: (0, 0)),
        ],
        out_shape=[
            jax.ShapeDtypeStruct((B, 16, N), jnp.float32),
            jax.ShapeDtypeStruct((8, Ch), jnp.float32),
        ],
    )(eh, es, vh, vs, s2w)


def _ap_body(eh_ref, vh_ref, al_ref, sm_ref, g_ref, bt_ref, o_ref, *, minv):
    m = sm_ref[0:1, :] * minv
    var = sm_ref[1:2, :] * minv - m * m
    scale = g_ref[...] * lax.rsqrt(var + _EPS)
    sh = bt_ref[...] - m * scale
    yh = eh_ref[0] + vh_ref[0][None]
    h = jnp.maximum(yh * scale[0] + sh[0], 0.0)
    al = al_ref[0, 0:_K, :]
    o_ref[0] = jnp.sum(h * al[..., None], axis=0)


def _apply(eh, vh, al, sm, g, bt, nb):
    B, _, N, Ch = eh.shape
    nch = N // nb
    minv = 1.0 / (B * _K * N)
    return pl.pallas_call(
        functools.partial(_ap_body, minv=minv),
        grid=(B, nch),
        in_specs=[
            pl.BlockSpec((1, _K, nb, Ch), lambda b, i: (b, 0, i, 0)),
            pl.BlockSpec((1, nb, Ch), lambda b, i: (b, i, 0)),
            pl.BlockSpec((1, 16, nb), lambda b, i: (b, 0, i)),
            pl.BlockSpec((8, Ch), lambda b, i: (0, 0)),
            pl.BlockSpec((1, Ch), lambda b, i: (0, 0)),
            pl.BlockSpec((1, Ch), lambda b, i: (0, 0)),
        ],
        out_specs=pl.BlockSpec((1, nb, Ch), lambda b, i: (b, i, 0)),
        out_shape=jax.ShapeDtypeStruct((B, N, Ch), jnp.float32),
    )(eh, vh, al, sm, g, bt)


def _head_body(f_ref, w1_ref, b1_ref, w2_ref, fcw_ref, fcb_ref, o_ref):
    f = f_ref[0]
    a1 = jnp.maximum(
        jnp.dot(f, w1_ref[...], preferred_element_type=jnp.float32)
        + b1_ref[...], 0.0)
    s = jnp.sum(a1 * w2_ref[...], axis=1, keepdims=True)
    mx = jnp.max(s, axis=0, keepdims=True)
    e = jnp.exp(s - mx)
    w = e / jnp.sum(e, axis=0, keepdims=True)
    pooled = lax.dot_general(w, f, (((0,), (0,)), ((), ())),
                             preferred_element_type=jnp.float32)
    out = jnp.maximum(
        jnp.dot(pooled, fcw_ref[...], preferred_element_type=jnp.float32)
        + fcb_ref[...], 0.0)
    o_ref[0] = jnp.broadcast_to(out, (8, out.shape[-1]))


def _head(f3, w1t, b1, w2, fcwt, fcb):
    B, N, C = f3.shape
    A = w1t.shape[1]
    return pl.pallas_call(
        _head_body,
        grid=(B,),
        in_specs=[
            pl.BlockSpec((1, N, C), lambda b: (b, 0, 0)),
            pl.BlockSpec((C, A), lambda b: (0, 0)),
            pl.BlockSpec((1, A), lambda b: (0, 0)),
            pl.BlockSpec((1, A), lambda b: (0, 0)),
            pl.BlockSpec((C, C), lambda b: (0, 0)),
            pl.BlockSpec((1, C), lambda b: (0, 0)),
        ],
        out_specs=pl.BlockSpec((1, 8, C), lambda b: (b, 0, 0)),
        out_shape=jax.ShapeDtypeStruct((B, 8, C), jnp.float32),
    )(f3, w1t, b1, w2, fcwt, fcb)[:, 0, :]


def kernel(x, ec1_w, ec1_g, ec1_b, ec2_mw, ec2_g, ec2_b, ec2_s1w, ec2_s1b,
           ec2_s2w, ec2_s2b, ec3_mw, ec3_g, ec3_b, ec3_s1w, ec3_s1b, ec3_s2w,
           ec3_s2b, att_w1, att_b1, att_w2, att_b2, fc_w, fc_b):
    B, N, _ = x.shape
    M = B * _K * N

    wg1 = jnp.concatenate([ec1_w[:, 0:3], ec1_w[:, 6:8]], axis=1)
    wc1 = jnp.concatenate([ec1_w[:, 3:6], ec1_w[:, 8:10]], axis=1)
    wg1t = jnp.pad(wg1.T, ((0, 0), (0, 64)))
    idx, U1, V1 = _knn_uv1(x, wg1t, (wc1 - wg1).T)

    E1 = _sc_gather1(idx, U1.reshape(B * N, -1), 0)
    f1 = _ep1(E1.reshape(B, _K, N, -1), V1, ec1_g[None], ec1_b[None])

    def soft_stage(T, mw, s1w, s1b, s2w, g, bt, nb):
        Cin = T.shape[-1]
        wgh, wch = mw[:, :Cin], mw[:, Cin:] - mw[:, :Cin]
        wgs, wcs = s1w[:, :Cin], s1w[:, Cin:] - s1w[:, :Cin]
        Cs = wgs.shape[0]
        if Cs % 128:
            p = 128 - Cs % 128
            wgs = jnp.pad(wgs, ((0, p), (0, 0)))
            wcs = jnp.pad(wcs, ((0, p), (0, 0)))
            s1b = jnp.pad(s1b, (0, p))
            s2w = jnp.pad(s2w, ((0, 0), (0, p)))
        Uh, Vh, Us, Vs = _uv(T, wgh.T, wch.T, wgs.T, wcs.T, s1b[None])
        Eh, Es = _sc_gather2(idx, Uh.reshape(B * N, -1),
                             Us.reshape(B * N, -1), 1)
        Eh = Eh.reshape(B, _K, N, -1)
        Es = Es.reshape(B, _K, N, -1)
        al, sm = _alpha_stats(Eh, Es, Vh, Vs, s2w, nb)
        return _apply(Eh, Vh, al, sm, g[None], bt[None], nb)

    f2 = soft_stage(f1, ec2_mw, ec2_s1w, ec2_s1b, ec2_s2w, ec2_g, ec2_b, 512)
    f3 = soft_stage(f2, ec3_mw, ec3_s1w, ec3_s1b, ec3_s2w, ec3_g, ec3_b, 512)

    return _head(f3, att_w1.T, att_b1[None], att_w2, fc_w.T, fc_b[None])


# speedup vs baseline: 4.8149x; 1.0449x over previous
import functools

import jax
import jax.numpy as jnp
from jax import lax
from jax.experimental import pallas as pl
from jax.experimental.pallas import tpu as pltpu
from jax.experimental.pallas import tpu_sc as plsc

_K = 10
_TAU = 0.2
_EPS = 1e-5


def _knn_uv1_body(x_ref, wg_ref, wc_ref, idx_ref, u_ref, v_ref):
    b = pl.program_id(0)
    N = x_ref.shape[1]
    t = x_ref[0]
    lane = lax.broadcasted_iota(jnp.int32, t.shape, 1)
    c = jnp.where(lane < 3, t, 0.0)
    sq = jnp.sum(c * c, axis=1, keepdims=True)
    dot = lax.dot_general(c, c, (((1,), (1,)), ((), ())),
                          preferred_element_type=jnp.float32)
    v = sq - 2.0 * dot
    row = lax.broadcasted_iota(jnp.int32, (N, N), 0)
    base = b * N
    for j in range(_K + 1):
        mn = jnp.min(v, axis=0, keepdims=True)
        am = jnp.min(jnp.where(v <= mn, row, N), axis=0, keepdims=True)
        idx_ref[0, j:j + 1, :] = am + base
        v = jnp.where(row == am, jnp.float32(1e30), v)
    idx_ref[0, _K + 1:16, :] = jnp.full((16 - _K - 1, N), base, jnp.int32)
    u_ref[0] = jnp.dot(t, wg_ref[...], preferred_element_type=jnp.float32)
    v_ref[0] = jnp.dot(t, wc_ref[...], preferred_element_type=jnp.float32)


def _knn_uv1(x, wg1t, wc1t):
    B, N, _ = x.shape
    Cu, Cv = wg1t.shape[1], wc1t.shape[1]
    return pl.pallas_call(
        _knn_uv1_body,
        grid=(B,),
        in_specs=[
            pl.BlockSpec((1, N, 5), lambda b: (b, 0, 0)),
            pl.BlockSpec((5, Cu), lambda b: (0, 0)),
            pl.BlockSpec((5, Cv), lambda b: (0, 0)),
        ],
        out_specs=[
            pl.BlockSpec((1, 16, N), lambda b: (b, 0, 0)),
            pl.BlockSpec((1, N, Cu), lambda b: (b, 0, 0)),
            pl.BlockSpec((1, N, Cv), lambda b: (b, 0, 0)),
        ],
        out_shape=[
            jax.ShapeDtypeStruct((B, 16, N), jnp.int32),
            jax.ShapeDtypeStruct((B, N, Cu), jnp.float32),
            jax.ShapeDtypeStruct((B, N, Cv), jnp.float32),
        ],
    )(x, wg1t, wc1t)


_CH = 128
_NW = 32


def _sc_gather(idx3, tb, j0):
    B = idx3.shape[0]
    C = tb.shape[1]
    N = tb.shape[0] // B
    M = B * _K * N
    per_w = (M // _CH) // _NW
    wpb = _NW // B
    mesh = plsc.VectorSubcoreMesh(core_axis_name="c", subcore_axis_name="s")

    @functools.partial(
        pl.kernel,
        mesh=mesh,
        out_type=jax.ShapeDtypeStruct((M, C), jnp.float32),
        scratch_types=[
            pltpu.VMEM((per_w * _CH,), jnp.int32),
            pltpu.VMEM((_CH, C), jnp.float32),
            pltpu.VMEM((_CH, C), jnp.float32),
            pltpu.SemaphoreType.DMA,
            pltpu.SemaphoreType.DMA,
            pltpu.SemaphoreType.DMA,
        ],
    )
    def k(idx_hbm, tb_hbm, o_hbm, idx_v, buf_a, buf_b, sg, sw0, sw1):
        wid = lax.axis_index("s") * 2 + lax.axis_index("c")
        b = wid // wpb
        woff = wid % wpb
        off0 = j0 * N + woff * per_w * _CH
        out0 = b * _K * N + woff * per_w * _CH
        pltpu.sync_copy(idx_hbm.at[b, pl.ds(off0, per_w * _CH)], idx_v)
        bufs = (buf_a, buf_b)
        sws = (sw0, sw1)

        def gstart(r):
            return pltpu.async_copy(
                tb_hbm.at[idx_v.at[pl.ds(r * _CH, _CH)]], bufs[r % 2], sg)

        def wstart(r):
            return pltpu.async_copy(
                bufs[r % 2], o_hbm.at[pl.ds(out0 + r * _CH, _CH), :],
                sws[r % 2])

        g_prev = gstart(0)
        w = [None] * per_w
        for r in range(1, per_w):
            g_prev.wait()
            w[r - 1] = wstart(r - 1)
            if r >= 2:
                w[r - 2].wait()
            g_prev = gstart(r)
        g_prev.wait()
        w[per_w - 1] = wstart(per_w - 1)
        w[per_w - 2].wait()
        w[per_w - 1].wait()

    return k(idx3, tb)


def _ep1_body(e_ref, v_ref, g_ref, bt_ref, wu_ref, wvh_ref, wvs_ref, sb_ref,
              u_ref, vh_ref, vs_ref):
    C = v_ref.shape[-1]
    y = e_ref[...][..., 0:C] + v_ref[...][:, None]
    y2 = y.reshape(-1, C)
    minv = 1.0 / y2.shape[0]
    m = jnp.sum(y2, axis=0, keepdims=True) * minv
    var = jnp.sum(y2 * y2, axis=0, keepdims=True) * minv - m * m
    scale = g_ref[...] * lax.rsqrt(var + _EPS)
    sh = bt_ref[...] - m * scale
    h = jnp.maximum(y * scale[0] + sh[0], 0.0)
    f = jnp.max(h, axis=1)
    B, N, _ = f.shape
    t = f.reshape(B * N, C)
    u_ref[...] = jnp.dot(t, wu_ref[...],
                         preferred_element_type=jnp.float32).reshape(
                             B, N, -1)
    vh_ref[...] = jnp.dot(t, wvh_ref[...],
                          preferred_element_type=jnp.float32).reshape(
                              B, N, -1)
    vs_ref[...] = (jnp.dot(t, wvs_ref[...],
                           preferred_element_type=jnp.float32)
                   + sb_ref[...]).reshape(B, N, -1)


def _ep1_uv2(e, v, g, bt, wu, wvh, wvs, sb):
    B, N, C = v.shape
    return pl.pallas_call(
        _ep1_body,
        out_shape=[
            jax.ShapeDtypeStruct((B, N, wu.shape[1]), jnp.float32),
            jax.ShapeDtypeStruct((B, N, wvh.shape[1]), jnp.float32),
            jax.ShapeDtypeStruct((B, N, wvs.shape[1]), jnp.float32),
        ],
    )(e, v, g, bt, wu, wvh, wvs, sb)


def _as_body(e_ref, vh_ref, vs_ref, w_ref, al_ref, sm_ref, *, Ch):
    nb = e_ref.shape[2]
    ys = e_ref[...][0, :, :, Ch:] + vs_ref[0][None]
    l = jnp.sum(jnp.maximum(ys, 0.0) * w_ref[...][0], axis=-1) * (1.0 / _TAU)
    mx = jnp.max(l, axis=0, keepdims=True)
    ex = jnp.exp(l - mx)
    al = ex / jnp.sum(ex, axis=0, keepdims=True)
    al_ref[0, 0:_K, :] = al
    al_ref[0, _K:16, :] = jnp.zeros((16 - _K, nb), jnp.float32)
    yh = e_ref[...][0, :, :, 0:Ch] + vh_ref[0][None]
    y2 = yh.reshape(-1, Ch)
    s1 = jnp.sum(y2, axis=0, keepdims=True)
    s2 = jnp.sum(y2 * y2, axis=0, keepdims=True)

    @pl.when(jnp.logical_and(pl.program_id(0) == 0, pl.program_id(1) == 0))
    def _():
        sm_ref[...] = jnp.zeros_like(sm_ref)

    sm_ref[0:1, :] += s1
    sm_ref[1:2, :] += s2


def _alpha_stats(e, vh, vs, s2w, nb, Ch):
    B, _, N, Cp = e.shape
    Cs = 128
    nch = N // nb
    return pl.pallas_call(
        functools.partial(_as_body, Ch=Ch),
        grid=(B, nch),
        in_specs=[
            pl.BlockSpec((1, _K, nb, Cp), lambda b, i: (b, 0, i, 0)),
            pl.BlockSpec((1, nb, Ch), lambda b, i: (b, i, 0)),
            pl.BlockSpec((1, nb, Cs), lambda b, i: (b, i, 0)),
            pl.BlockSpec((1, Cs), lambda b, i: (0, 0)),
        ],
        out_specs=[
            pl.BlockSpec((1, 16, nb), lambda b, i: (b, 0, i)),
            pl.BlockSpec((8, Ch), lambda b, i: (0, 0)),
        ],
        out_shape=[
            jax.ShapeDtypeStruct((B, 16, N), jnp.float32),
            jax.ShapeDtypeStruct((8, Ch), jnp.float32),
        ],
    )(e, vh, vs, s2w)


def _ap_f(eh_ref, vh_ref, al_ref, sm_ref, g_ref, bt_ref, *, minv, Ch):
    m = sm_ref[0:1, :] * minv
    var = sm_ref[1:2, :] * minv - m * m
    scale = g_ref[...] * lax.rsqrt(var + _EPS)
    sh = bt_ref[...] - m * scale
    yh = eh_ref[...][0, :, :, 0:Ch] + vh_ref[0][None]
    h = jnp.maximum(yh * scale[0] + sh[0], 0.0)
    al = al_ref[0, 0:_K, :]
    return jnp.sum(h * al[..., None], axis=0)


def _ap_out_body(eh_ref, vh_ref, al_ref, sm_ref, g_ref, bt_ref, o_ref,
                 *, minv, Ch):
    o_ref[0] = _ap_f(eh_ref, vh_ref, al_ref, sm_ref, g_ref, bt_ref,
                     minv=minv, Ch=Ch)


def _ap_uv_body(eh_ref, vh_ref, al_ref, sm_ref, g_ref, bt_ref,
                wu_ref, wvh_ref, wvs_ref, sb_ref,
                u_ref, uvh_ref, uvs_ref, *, minv, Ch):
    f = _ap_f(eh_ref, vh_ref, al_ref, sm_ref, g_ref, bt_ref,
              minv=minv, Ch=Ch)
    u_ref[0] = jnp.dot(f, wu_ref[...], preferred_element_type=jnp.float32)
    uvh_ref[0] = jnp.dot(f, wvh_ref[...], preferred_element_type=jnp.float32)
    uvs_ref[0] = (jnp.dot(f, wvs_ref[...], preferred_element_type=jnp.float32)
                  + sb_ref[...])


def _common_specs(B, N, Cp, Ch, nb):
    return [
        pl.BlockSpec((1, _K, nb, Cp), lambda b, i: (b, 0, i, 0)),
        pl.BlockSpec((1, nb, Ch), lambda b, i: (b, i, 0)),
        pl.BlockSpec((1, 16, nb), lambda b, i: (b, 0, i)),
        pl.BlockSpec((8, Ch), lambda b, i: (0, 0)),
        pl.BlockSpec((1, Ch), lambda b, i: (0, 0)),
        pl.BlockSpec((1, Ch), lambda b, i: (0, 0)),
    ]


def _apply_out(e, vh, al, sm, g, bt, nb, Ch):
    B, _, N, Cp = e.shape
    nch = N // nb
    minv = 1.0 / (B * _K * N)
    return pl.pallas_call(
        functools.partial(_ap_out_body, minv=minv, Ch=Ch),
        grid=(B, nch),
        in_specs=_common_specs(B, N, Cp, Ch, nb),
        out_specs=pl.BlockSpec((1, nb, Ch), lambda b, i: (b, i, 0)),
        out_shape=jax.ShapeDtypeStruct((B, N, Ch), jnp.float32),
    )(e, vh, al, sm, g, bt)


def _apply_uv(e, vh, al, sm, g, bt, wu, wvh, wvs, sb, nb, Ch):
    B, _, N, Cp = e.shape
    nch = N // nb
    minv = 1.0 / (B * _K * N)
    Cu, Cvh, Cvs = wu.shape[1], wvh.shape[1], wvs.shape[1]
    specs = _common_specs(B, N, Cp, Ch, nb) + [
        pl.BlockSpec((Ch, Cu), lambda b, i: (0, 0)),
        pl.BlockSpec((Ch, Cvh), lambda b, i: (0, 0)),
        pl.BlockSpec((Ch, Cvs), lambda b, i: (0, 0)),
        pl.BlockSpec((1, Cvs), lambda b, i: (0, 0)),
    ]
    return pl.pallas_call(
        functools.partial(_ap_uv_body, minv=minv, Ch=Ch),
        grid=(B, nch),
        in_specs=specs,
        out_specs=[
            pl.BlockSpec((1, nb, Cu), lambda b, i: (b, i, 0)),
            pl.BlockSpec((1, nb, Cvh), lambda b, i: (b, i, 0)),
            pl.BlockSpec((1, nb, Cvs), lambda b, i: (b, i, 0)),
        ],
        out_shape=[
            jax.ShapeDtypeStruct((B, N, Cu), jnp.float32),
            jax.ShapeDtypeStruct((B, N, Cvh), jnp.float32),
            jax.ShapeDtypeStruct((B, N, Cvs), jnp.float32),
        ],
    )(e, vh, al, sm, g, bt, wu, wvh, wvs, sb)


def _head_body(f_ref, w1_ref, b1_ref, w2_ref, fcw_ref, fcb_ref, o_ref):
    f = f_ref[0]
    a1 = jnp.maximum(
        jnp.dot(f, w1_ref[...], preferred_element_type=jnp.float32)
        + b1_ref[...], 0.0)
    s = jnp.sum(a1 * w2_ref[...], axis=1, keepdims=True)
    mx = jnp.max(s, axis=0, keepdims=True)
    e = jnp.exp(s - mx)
    w = e / jnp.sum(e, axis=0, keepdims=True)
    pooled = lax.dot_general(w, f, (((0,), (0,)), ((), ())),
                             preferred_element_type=jnp.float32)
    out = jnp.maximum(
        jnp.dot(pooled, fcw_ref[...], preferred_element_type=jnp.float32)
        + fcb_ref[...], 0.0)
    o_ref[0] = jnp.broadcast_to(out, (8, out.shape[-1]))


def _head(f3, w1t, b1, w2, fcwt, fcb):
    B, N, C = f3.shape
    A = w1t.shape[1]
    return pl.pallas_call(
        _head_body,
        grid=(B,),
        in_specs=[
            pl.BlockSpec((1, N, C), lambda b: (b, 0, 0)),
            pl.BlockSpec((C, A), lambda b: (0, 0)),
            pl.BlockSpec((1, A), lambda b: (0, 0)),
            pl.BlockSpec((1, A), lambda b: (0, 0)),
            pl.BlockSpec((C, C), lambda b: (0, 0)),
            pl.BlockSpec((1, C), lambda b: (0, 0)),
        ],
        out_specs=pl.BlockSpec((1, 8, C), lambda b: (b, 0, 0)),
        out_shape=jax.ShapeDtypeStruct((B, 8, C), jnp.float32),
    )(f3, w1t, b1, w2, fcwt, fcb)[:, 0, :]


def kernel(x, ec1_w, ec1_g, ec1_b, ec2_mw, ec2_g, ec2_b, ec2_s1w, ec2_s1b,
           ec2_s2w, ec2_s2b, ec3_mw, ec3_g, ec3_b, ec3_s1w, ec3_s1b, ec3_s2w,
           ec3_s2b, att_w1, att_b1, att_w2, att_b2, fc_w, fc_b):
    B, N, _ = x.shape
    M = B * _K * N

    wg1 = jnp.concatenate([ec1_w[:, 0:3], ec1_w[:, 6:8]], axis=1)
    wc1 = jnp.concatenate([ec1_w[:, 3:6], ec1_w[:, 8:10]], axis=1)
    wg1t = jnp.pad(wg1.T, ((0, 0), (0, 64)))
    idx, U1, V1 = _knn_uv1(x, wg1t, (wc1 - wg1).T)
    idx3 = idx.reshape(B, 16 * N)

    def split(mw, s1w, s1b, s2w, Cin):
        wgh, wch = mw[:, :Cin], mw[:, Cin:] - mw[:, :Cin]
        wgs, wcs = s1w[:, :Cin], s1w[:, Cin:] - s1w[:, :Cin]
        Cs = wgs.shape[0]
        if Cs % 128:
            p = 128 - Cs % 128
            wgs = jnp.pad(wgs, ((0, p), (0, 0)))
            wcs = jnp.pad(wcs, ((0, p), (0, 0)))
            s1b = jnp.pad(s1b, (0, p))
            s2w = jnp.pad(s2w, ((0, 0), (0, p)))
        wu = jnp.concatenate([wgh.T, wgs.T], axis=1)
        return wu, wch.T, wcs.T, s1b[None], s2w

    wu2, wvh2, wvs2, sb2, s2w2 = split(ec2_mw, ec2_s1w, ec2_s1b, ec2_s2w, 64)
    wu3, wvh3, wvs3, sb3, s2w3 = split(ec3_mw, ec3_s1w, ec3_s1b, ec3_s2w, 128)

    E1 = _sc_gather(idx3, U1.reshape(B * N, -1), 0)
    U2, V2h, V2s = _ep1_uv2(E1.reshape(B, _K, N, -1), V1,
                            ec1_g[None], ec1_b[None], wu2, wvh2, wvs2, sb2)

    E2 = _sc_gather(idx3, U2.reshape(B * N, -1), 1).reshape(B, _K, N, -1)
    al2, sm2 = _alpha_stats(E2, V2h, V2s, s2w2, 512, 128)
    U3, V3h, V3s = _apply_uv(E2, V2h, al2, sm2, ec2_g[None], ec2_b[None],
                             wu3, wvh3, wvs3, sb3, 512, 128)

    E3 = _sc_gather(idx3, U3.reshape(B * N, -1), 1).reshape(B, _K, N, -1)
    al3, sm3 = _alpha_stats(E3, V3h, V3s, s2w3, 512, 256)
    f3 = _apply_out(E3, V3h, al3, sm3, ec3_g[None], ec3_b[None], 512, 256)

    return _head(f3, att_w1.T, att_b1[None], att_w2, fc_w.T, fc_b[None])


# speedup vs baseline: 4.8837x; 1.0143x over previous
import functools

import jax
import jax.numpy as jnp
from jax import lax
from jax.experimental import pallas as pl
from jax.experimental.pallas import tpu as pltpu
from jax.experimental.pallas import tpu_sc as plsc

_K = 10
_TAU = 0.2
_EPS = 1e-5


def _knn_uv1_body(x_ref, wg_ref, wc_ref, idx_ref, u_ref, v_ref):
    b = pl.program_id(0)
    N = x_ref.shape[1]
    t = x_ref[0]
    lane = lax.broadcasted_iota(jnp.int32, t.shape, 1)
    c = jnp.where(lane < 3, t, 0.0)
    sq = jnp.sum(c * c, axis=1, keepdims=True)
    dot = lax.dot_general(c, c, (((1,), (1,)), ((), ())),
                          preferred_element_type=jnp.float32)
    row = lax.broadcasted_iota(jnp.int32, (N, N), 0)
    col = lax.broadcasted_iota(jnp.int32, (N, N), 1)
    v = jnp.where(row == col, jnp.float32(1e30), sq - 2.0 * dot)
    base = b * N
    idx_ref[0, 0:1, :] = lax.broadcasted_iota(jnp.int32, (1, N), 1) + base
    for j in range(1, _K + 1):
        mn = jnp.min(v, axis=0, keepdims=True)
        am = jnp.min(jnp.where(v <= mn, row, N), axis=0, keepdims=True)
        idx_ref[0, j:j + 1, :] = am + base
        v = jnp.where(row == am, jnp.float32(1e30), v)
    idx_ref[0, _K + 1:16, :] = jnp.full((16 - _K - 1, N), base, jnp.int32)
    u_ref[0] = jnp.dot(t, wg_ref[...], preferred_element_type=jnp.float32)
    v_ref[0] = jnp.dot(t, wc_ref[...], preferred_element_type=jnp.float32)


def _knn_uv1(x, wg1t, wc1t):
    B, N, _ = x.shape
    Cu, Cv = wg1t.shape[1], wc1t.shape[1]
    return pl.pallas_call(
        _knn_uv1_body,
        grid=(B,),
        in_specs=[
            pl.BlockSpec((1, N, 5), lambda b: (b, 0, 0)),
            pl.BlockSpec((5, Cu), lambda b: (0, 0)),
            pl.BlockSpec((5, Cv), lambda b: (0, 0)),
        ],
        out_specs=[
            pl.BlockSpec((1, 16, N), lambda b: (b, 0, 0)),
            pl.BlockSpec((1, N, Cu), lambda b: (b, 0, 0)),
            pl.BlockSpec((1, N, Cv), lambda b: (b, 0, 0)),
        ],
        out_shape=[
            jax.ShapeDtypeStruct((B, 16, N), jnp.int32),
            jax.ShapeDtypeStruct((B, N, Cu), jnp.float32),
            jax.ShapeDtypeStruct((B, N, Cv), jnp.float32),
        ],
    )(x, wg1t, wc1t)


_CH = 128
_NW = 32


def _sc_gather(idx3, tb, j0):
    B = idx3.shape[0]
    C = tb.shape[1]
    N = tb.shape[0] // B
    M = B * _K * N
    per_w = (M // _CH) // _NW
    wpb = _NW // B
    mesh = plsc.VectorSubcoreMesh(core_axis_name="c", subcore_axis_name="s")

    @functools.partial(
        pl.kernel,
        mesh=mesh,
        out_type=jax.ShapeDtypeStruct((M, C), jnp.float32),
        scratch_types=[
            pltpu.VMEM((per_w * _CH,), jnp.int32),
            pltpu.VMEM((_CH, C), jnp.float32),
            pltpu.VMEM((_CH, C), jnp.float32),
            pltpu.SemaphoreType.DMA,
            pltpu.SemaphoreType.DMA,
            pltpu.SemaphoreType.DMA,
        ],
    )
    def k(idx_hbm, tb_hbm, o_hbm, idx_v, buf_a, buf_b, sg, sw0, sw1):
        wid = lax.axis_index("s") * 2 + lax.axis_index("c")
        b = wid // wpb
        woff = wid % wpb
        off0 = j0 * N + woff * per_w * _CH
        out0 = b * _K * N + woff * per_w * _CH
        pltpu.sync_copy(idx_hbm.at[b, pl.ds(off0, per_w * _CH)], idx_v)
        bufs = (buf_a, buf_b)
        sws = (sw0, sw1)

        def gstart(r):
            return pltpu.async_copy(
                tb_hbm.at[idx_v.at[pl.ds(r * _CH, _CH)]], bufs[r % 2], sg)

        def wstart(r):
            return pltpu.async_copy(
                bufs[r % 2], o_hbm.at[pl.ds(out0 + r * _CH, _CH), :],
                sws[r % 2])

        g_prev = gstart(0)
        w = [None] * per_w
        for r in range(1, per_w):
            g_prev.wait()
            w[r - 1] = wstart(r - 1)
            if r >= 2:
                w[r - 2].wait()
            g_prev = gstart(r)
        g_prev.wait()
        w[per_w - 1] = wstart(per_w - 1)
        w[per_w - 2].wait()
        w[per_w - 1].wait()

    return k(idx3, tb)


def _ep1_body(e_ref, v_ref, g_ref, bt_ref, wu_ref, wvh_ref, wvs_ref, sb_ref,
              sm_ref, u_ref, vh_ref, vs_ref, *, minv):
    s = pl.program_id(0)
    C = v_ref.shape[-1]

    @pl.when(s == 0)
    def _():
        y = e_ref[...][0, :, :, 0:C] + v_ref[0][None]
        y2 = y.reshape(-1, C)
        ones = jnp.ones((1, y2.shape[0]), jnp.float32)
        s1 = lax.dot_general(ones, y2, (((1,), (0,)), ((), ())),
                             preferred_element_type=jnp.float32)
        s2 = lax.dot_general(ones, y2 * y2, (((1,), (0,)), ((), ())),
                             preferred_element_type=jnp.float32)

        @pl.when(jnp.logical_and(pl.program_id(1) == 0, pl.program_id(2) == 0))
        def _():
            sm_ref[...] = jnp.zeros_like(sm_ref)

        sm_ref[0:1, :] += s1
        sm_ref[1:2, :] += s2

    @pl.when(s == 1)
    def _():
        y = e_ref[...][0, :, :, 0:C] + v_ref[0][None]
        m = sm_ref[0:1, :] * minv
        var = sm_ref[1:2, :] * minv - m * m
        scale = g_ref[...] * lax.rsqrt(var + _EPS)
        sh = bt_ref[...] - m * scale
        h = jnp.maximum(y * scale[0] + sh[0], 0.0)
        f = jnp.max(h, axis=0)
        u_ref[0] = jnp.dot(f, wu_ref[...],
                           preferred_element_type=jnp.float32)
        vh_ref[0] = jnp.dot(f, wvh_ref[...],
                            preferred_element_type=jnp.float32)
        vs_ref[0] = (jnp.dot(f, wvs_ref[...],
                             preferred_element_type=jnp.float32)
                     + sb_ref[...])


def _ep1_uv2(e, v, g, bt, wu, wvh, wvs, sb, nb=512):
    B, N, C = v.shape
    Cp = e.shape[-1]
    nch = N // nb
    minv = 1.0 / (B * _K * N)
    Cu, Cvh, Cvs = wu.shape[1], wvh.shape[1], wvs.shape[1]
    return pl.pallas_call(
        functools.partial(_ep1_body, minv=minv),
        grid=(2, B, nch),
        in_specs=[
            pl.BlockSpec((1, _K, nb, Cp), lambda s, b, i: (b, 0, i, 0)),
            pl.BlockSpec((1, nb, C), lambda s, b, i: (b, i, 0)),
            pl.BlockSpec((1, C), lambda s, b, i: (0, 0)),
            pl.BlockSpec((1, C), lambda s, b, i: (0, 0)),
            pl.BlockSpec((C, Cu), lambda s, b, i: (0, 0)),
            pl.BlockSpec((C, Cvh), lambda s, b, i: (0, 0)),
            pl.BlockSpec((C, Cvs), lambda s, b, i: (0, 0)),
            pl.BlockSpec((1, Cvs), lambda s, b, i: (0, 0)),
        ],
        out_specs=[
            pl.BlockSpec((8, C), lambda s, b, i: (0, 0)),
            pl.BlockSpec((1, nb, Cu), lambda s, b, i: (b, i, 0)),
            pl.BlockSpec((1, nb, Cvh), lambda s, b, i: (b, i, 0)),
            pl.BlockSpec((1, nb, Cvs), lambda s, b, i: (b, i, 0)),
        ],
        out_shape=[
            jax.ShapeDtypeStruct((8, C), jnp.float32),
            jax.ShapeDtypeStruct((B, N, Cu), jnp.float32),
            jax.ShapeDtypeStruct((B, N, Cvh), jnp.float32),
            jax.ShapeDtypeStruct((B, N, Cvs), jnp.float32),
        ],
    )(e, v, g, bt, wu, wvh, wvs, sb)[1:]


def _alpha_of(ee, vs_ref, w_ref, Ch):
    ys = ee[0, :, :, Ch:] + vs_ref[0][None]
    l = jnp.sum(jnp.maximum(ys, 0.0) * w_ref[...][0], axis=-1) * (1.0 / _TAU)
    mx = jnp.max(l, axis=0, keepdims=True)
    ex = jnp.exp(l - mx)
    return ex / jnp.sum(ex, axis=0, keepdims=True)


def _stats_accum(ee, vh_ref, sm_ref, first, Ch):
    yh = ee[0, :, :, 0:Ch] + vh_ref[0][None]
    y2 = yh.reshape(-1, Ch)
    ones = jnp.ones((1, y2.shape[0]), jnp.float32)
    s1 = lax.dot_general(ones, y2, (((1,), (0,)), ((), ())),
                         preferred_element_type=jnp.float32)
    s2 = lax.dot_general(ones, y2 * y2, (((1,), (0,)), ((), ())),
                         preferred_element_type=jnp.float32)

    @pl.when(first)
    def _():
        sm_ref[...] = jnp.zeros_like(sm_ref)

    sm_ref[0:1, :] += s1
    sm_ref[1:2, :] += s2


def _combine(ee, vh_ref, vs_ref, w_ref, sm_ref, g_ref, bt_ref, minv, Ch):
    al = _alpha_of(ee, vs_ref, w_ref, Ch)
    m = sm_ref[0:1, :] * minv
    var = sm_ref[1:2, :] * minv - m * m
    scale = g_ref[...] * lax.rsqrt(var + _EPS)
    sh = bt_ref[...] - m * scale
    yh = ee[0, :, :, 0:Ch] + vh_ref[0][None]
    h = jnp.maximum(yh * scale[0] + sh[0], 0.0)
    return jnp.sum(h * al[..., None], axis=0)


def _st2_body(e_ref, vh_ref, vs_ref, w_ref, g_ref, bt_ref,
              wu_ref, wvh_ref, wvs_ref, sb_ref,
              sm_ref, u_ref, uvh_ref, uvs_ref, *, minv, Ch):
    s = pl.program_id(0)

    @pl.when(s == 0)
    def _():
        ee = e_ref[...]
        first = jnp.logical_and(pl.program_id(1) == 0, pl.program_id(2) == 0)
        _stats_accum(ee, vh_ref, sm_ref, first, Ch)

    @pl.when(s == 1)
    def _():
        ee = e_ref[...]
        f = _combine(ee, vh_ref, vs_ref, w_ref, sm_ref, g_ref, bt_ref,
                     minv, Ch)
        u_ref[0] = jnp.dot(f, wu_ref[...],
                           preferred_element_type=jnp.float32)
        uvh_ref[0] = jnp.dot(f, wvh_ref[...],
                             preferred_element_type=jnp.float32)
        uvs_ref[0] = (jnp.dot(f, wvs_ref[...],
                              preferred_element_type=jnp.float32)
                      + sb_ref[...])


def _stage2(e, vh, vs, s2w, g, bt, wu, wvh, wvs, sb, nb, Ch):
    B, _, N, Cp = e.shape
    nch = N // nb
    minv = 1.0 / (B * _K * N)
    Cu, Cvh, Cvs = wu.shape[1], wvh.shape[1], wvs.shape[1]
    specs = [
        pl.BlockSpec((1, _K, nb, Cp), lambda s, b, i: (b, 0, i, 0)),
        pl.BlockSpec((1, nb, Ch), lambda s, b, i: (b, i, 0)),
        pl.BlockSpec((1, nb, 128), lambda s, b, i: (b, i, 0)),
        pl.BlockSpec((1, 128), lambda s, b, i: (0, 0)),
        pl.BlockSpec((1, Ch), lambda s, b, i: (0, 0)),
        pl.BlockSpec((1, Ch), lambda s, b, i: (0, 0)),
        pl.BlockSpec((Ch, Cu), lambda s, b, i: (0, 0)),
        pl.BlockSpec((Ch, Cvh), lambda s, b, i: (0, 0)),
        pl.BlockSpec((Ch, Cvs), lambda s, b, i: (0, 0)),
        pl.BlockSpec((1, Cvs), lambda s, b, i: (0, 0)),
    ]
    return pl.pallas_call(
        functools.partial(_st2_body, minv=minv, Ch=Ch),
        grid=(2, B, nch),
        in_specs=specs,
        out_specs=[
            pl.BlockSpec((8, Ch), lambda s, b, i: (0, 0)),
            pl.BlockSpec((1, nb, Cu), lambda s, b, i: (b, i, 0)),
            pl.BlockSpec((1, nb, Cvh), lambda s, b, i: (b, i, 0)),
            pl.BlockSpec((1, nb, Cvs), lambda s, b, i: (b, i, 0)),
        ],
        out_shape=[
            jax.ShapeDtypeStruct((8, Ch), jnp.float32),
            jax.ShapeDtypeStruct((B, N, Cu), jnp.float32),
            jax.ShapeDtypeStruct((B, N, Cvh), jnp.float32),
            jax.ShapeDtypeStruct((B, N, Cvs), jnp.float32),
        ],
    )(e, vh, vs, s2w, g, bt, wu, wvh, wvs, sb)[1:]


def _st3_body(e_ref, vh_ref, vs_ref, w_ref, g_ref, bt_ref,
              w1_ref, b1_ref, w2_ref, fcw_ref, fcb_ref,
              sm_ref, o_ref, *, minv, Ch):
    s = pl.program_id(0)

    @pl.when(s == 0)
    def _():
        ee = e_ref[...]
        _stats_accum(ee, vh_ref, sm_ref, pl.program_id(1) == 0, Ch)

    @pl.when(s == 1)
    def _():
        ee = e_ref[...]
        f = _combine(ee, vh_ref, vs_ref, w_ref, sm_ref, g_ref, bt_ref,
                     minv, Ch)
        a1 = jnp.maximum(
            jnp.dot(f, w1_ref[...], preferred_element_type=jnp.float32)
            + b1_ref[...], 0.0)
        sc = jnp.sum(a1 * w2_ref[...], axis=1, keepdims=True)
        mx = jnp.max(sc, axis=0, keepdims=True)
        ex = jnp.exp(sc - mx)
        w = ex / jnp.sum(ex, axis=0, keepdims=True)
        pooled = lax.dot_general(w, f, (((0,), (0,)), ((), ())),
                                 preferred_element_type=jnp.float32)
        out = jnp.maximum(
            jnp.dot(pooled, fcw_ref[...], preferred_element_type=jnp.float32)
            + fcb_ref[...], 0.0)
        o_ref[0] = jnp.broadcast_to(out, (8, out.shape[-1]))


def _stage3_head(e, vh, vs, s2w, g, bt, w1t, b1, w2, fcwt, fcb):
    B, _, N, Cp = e.shape
    Ch = vh.shape[-1]
    A = w1t.shape[1]
    minv = 1.0 / (B * _K * N)
    specs = [
        pl.BlockSpec((1, _K, N, Cp), lambda s, b: (b, 0, 0, 0)),
        pl.BlockSpec((1, N, Ch), lambda s, b: (b, 0, 0)),
        pl.BlockSpec((1, N, 128), lambda s, b: (b, 0, 0)),
        pl.BlockSpec((1, 128), lambda s, b: (0, 0)),
        pl.BlockSpec((1, Ch), lambda s, b: (0, 0)),
        pl.BlockSpec((1, Ch), lambda s, b: (0, 0)),
        pl.BlockSpec((Ch, A), lambda s, b: (0, 0)),
        pl.BlockSpec((1, A), lambda s, b: (0, 0)),
        pl.BlockSpec((1, A), lambda s, b: (0, 0)),
        pl.BlockSpec((Ch, Ch), lambda s, b: (0, 0)),
        pl.BlockSpec((1, Ch), lambda s, b: (0, 0)),
    ]
    return pl.pallas_call(
        functools.partial(_st3_body, minv=minv, Ch=Ch),
        grid=(2, B),
        in_specs=specs,
        out_specs=[
            pl.BlockSpec((8, Ch), lambda s, b: (0, 0)),
            pl.BlockSpec((1, 8, Ch), lambda s, b: (b, 0, 0)),
        ],
        out_shape=[
            jax.ShapeDtypeStruct((8, Ch), jnp.float32),
            jax.ShapeDtypeStruct((B, 8, Ch), jnp.float32),
        ],
    )(e, vh, vs, s2w, g, bt, w1t, b1, w2, fcwt, fcb)[1][:, 0, :]


def kernel(x, ec1_w, ec1_g, ec1_b, ec2_mw, ec2_g, ec2_b, ec2_s1w, ec2_s1b,
           ec2_s2w, ec2_s2b, ec3_mw, ec3_g, ec3_b, ec3_s1w, ec3_s1b, ec3_s2w,
           ec3_s2b, att_w1, att_b1, att_w2, att_b2, fc_w, fc_b):
    B, N, _ = x.shape
    M = B * _K * N

    wg1 = jnp.concatenate([ec1_w[:, 0:3], ec1_w[:, 6:8]], axis=1)
    wc1 = jnp.concatenate([ec1_w[:, 3:6], ec1_w[:, 8:10]], axis=1)
    wg1t = jnp.pad(wg1.T, ((0, 0), (0, 64)))
    idx, U1, V1 = _knn_uv1(x, wg1t, (wc1 - wg1).T)
    idx3 = idx.reshape(B, 16 * N)

    def split(mw, s1w, s1b, s2w, Cin):
        wgh, wch = mw[:, :Cin], mw[:, Cin:] - mw[:, :Cin]
        wgs, wcs = s1w[:, :Cin], s1w[:, Cin:] - s1w[:, :Cin]
        Cs = wgs.shape[0]
        if Cs % 128:
            p = 128 - Cs % 128
            wgs = jnp.pad(wgs, ((0, p), (0, 0)))
            wcs = jnp.pad(wcs, ((0, p), (0, 0)))
            s1b = jnp.pad(s1b, (0, p))
            s2w = jnp.pad(s2w, ((0, 0), (0, p)))
        wu = jnp.concatenate([wgh.T, wgs.T], axis=1)
        return wu, wch.T, wcs.T, s1b[None], s2w

    wu2, wvh2, wvs2, sb2, s2w2 = split(ec2_mw, ec2_s1w, ec2_s1b, ec2_s2w, 64)
    wu3, wvh3, wvs3, sb3, s2w3 = split(ec3_mw, ec3_s1w, ec3_s1b, ec3_s2w, 128)

    E1 = _sc_gather(idx3, U1.reshape(B * N, -1), 0)
    U2, V2h, V2s = _ep1_uv2(E1.reshape(B, _K, N, -1), V1,
                            ec1_g[None], ec1_b[None], wu2, wvh2, wvs2, sb2)

    E2 = _sc_gather(idx3, U2.reshape(B * N, -1), 1).reshape(B, _K, N, -1)
    U3, V3h, V3s = _stage2(E2, V2h, V2s, s2w2, ec2_g[None], ec2_b[None],
                           wu3, wvh3, wvs3, sb3, 512, 128)

    E3 = _sc_gather(idx3, U3.reshape(B * N, -1), 1).reshape(B, _K, N, -1)
    return _stage3_head(E3, V3h, V3s, s2w3, ec3_g[None], ec3_b[None],
                        att_w1.T, att_b1[None], att_w2, fc_w.T, fc_b[None])


# speedup vs baseline: 5.0053x; 1.0249x over previous
import functools

import jax
import jax.numpy as jnp
from jax import lax
from jax.experimental import pallas as pl
from jax.experimental.pallas import tpu as pltpu
from jax.experimental.pallas import tpu_sc as plsc

_K = 10
_TAU = 0.2
_EPS = 1e-5


def _knn_uv1_body(x_ref, wg_ref, wc_ref, idx_ref, u_ref, v_ref):
    b = pl.program_id(0)
    N = x_ref.shape[1]
    t = x_ref[0]
    lane = lax.broadcasted_iota(jnp.int32, t.shape, 1)
    c = jnp.where(lane < 3, t, 0.0)
    sq = jnp.sum(c * c, axis=1, keepdims=True)
    dot = lax.dot_general(c, c, (((1,), (1,)), ((), ())),
                          preferred_element_type=jnp.float32)
    row = lax.broadcasted_iota(jnp.int32, (N, N), 0)
    col = lax.broadcasted_iota(jnp.int32, (N, N), 1)
    v = jnp.where(row == col, jnp.float32(1e30), sq - 2.0 * dot)
    base = b * N
    idx_ref[0, 0:1, :] = lax.broadcasted_iota(jnp.int32, (1, N), 1) + base
    for j in range(1, _K + 1):
        mn = jnp.min(v, axis=0, keepdims=True)
        am = jnp.min(jnp.where(v <= mn, row, N), axis=0, keepdims=True)
        idx_ref[0, j:j + 1, :] = am + base
        v = jnp.where(row == am, jnp.float32(1e30), v)
    idx_ref[0, _K + 1:16, :] = jnp.full((16 - _K - 1, N), base, jnp.int32)
    u_ref[0] = jnp.dot(t, wg_ref[...], preferred_element_type=jnp.float32)
    v_ref[0] = jnp.dot(t, wc_ref[...], preferred_element_type=jnp.float32)


def _knn_uv1(x, wg1t, wc1t):
    B, N, _ = x.shape
    Cu, Cv = wg1t.shape[1], wc1t.shape[1]
    return pl.pallas_call(
        _knn_uv1_body,
        grid=(B,),
        in_specs=[
            pl.BlockSpec((1, N, 5), lambda b: (b, 0, 0)),
            pl.BlockSpec((5, Cu), lambda b: (0, 0)),
            pl.BlockSpec((5, Cv), lambda b: (0, 0)),
        ],
        out_specs=[
            pl.BlockSpec((1, 16, N), lambda b: (b, 0, 0)),
            pl.BlockSpec((1, N, Cu), lambda b: (b, 0, 0)),
            pl.BlockSpec((1, N, Cv), lambda b: (b, 0, 0)),
        ],
        out_shape=[
            jax.ShapeDtypeStruct((B, 16, N), jnp.int32),
            jax.ShapeDtypeStruct((B, N, Cu), jnp.float32),
            jax.ShapeDtypeStruct((B, N, Cv), jnp.float32),
        ],
    )(x, wg1t, wc1t)


_CH = 128
_NW = 32


def _sc_gather(idx3, tb, j0):
    B = idx3.shape[0]
    C = tb.shape[1]
    N = tb.shape[0] // B
    M = B * _K * N
    per_w = (M // _CH) // _NW
    wpb = _NW // B
    mesh = plsc.VectorSubcoreMesh(core_axis_name="c", subcore_axis_name="s")

    @functools.partial(
        pl.kernel,
        mesh=mesh,
        out_type=jax.ShapeDtypeStruct((M, C), jnp.float32),
        scratch_types=[
            pltpu.VMEM((per_w * _CH,), jnp.int32),
            pltpu.VMEM((_CH, C), jnp.float32),
            pltpu.VMEM((_CH, C), jnp.float32),
            pltpu.SemaphoreType.DMA,
            pltpu.SemaphoreType.DMA,
            pltpu.SemaphoreType.DMA,
        ],
    )
    def k(idx_hbm, tb_hbm, o_hbm, idx_v, buf_a, buf_b, sg, sw0, sw1):
        wid = lax.axis_index("s") * 2 + lax.axis_index("c")
        b = wid // wpb
        woff = wid % wpb
        off0 = j0 * N + woff * per_w * _CH
        out0 = b * _K * N + woff * per_w * _CH
        pltpu.sync_copy(idx_hbm.at[b, pl.ds(off0, per_w * _CH)], idx_v)
        bufs = (buf_a, buf_b)
        sws = (sw0, sw1)

        def gstart(r):
            return pltpu.async_copy(
                tb_hbm.at[idx_v.at[pl.ds(r * _CH, _CH)]], bufs[r % 2], sg)

        def wstart(r):
            return pltpu.async_copy(
                bufs[r % 2], o_hbm.at[pl.ds(out0 + r * _CH, _CH), :],
                sws[r % 2])

        g_prev = gstart(0)
        w = [None] * per_w
        for r in range(1, per_w):
            g_prev.wait()
            w[r - 1] = wstart(r - 1)
            if r >= 2:
                w[r - 2].wait()
            g_prev = gstart(r)
        g_prev.wait()
        w[per_w - 1] = wstart(per_w - 1)
        w[per_w - 2].wait()
        w[per_w - 1].wait()

    return k(idx3, tb)


def _ep1_body(e_ref, v_ref, g_ref, bt_ref, wu_ref, wvh_ref, wvs_ref, sb_ref,
              u_ref, vh_ref, vs_ref):
    C = v_ref.shape[-1]
    y = e_ref[...][..., 0:C] + v_ref[...][:, None]
    y2 = y.reshape(-1, C)
    minv = 1.0 / y2.shape[0]
    ones = jnp.ones((1, y2.shape[0]), jnp.float32)
    m = lax.dot_general(ones, y2, (((1,), (0,)), ((), ())),
                        preferred_element_type=jnp.float32) * minv
    var = lax.dot_general(ones, y2 * y2, (((1,), (0,)), ((), ())),
                          preferred_element_type=jnp.float32) * minv - m * m
    scale = g_ref[...] * lax.rsqrt(var + _EPS)
    sh = bt_ref[...] - m * scale
    h = jnp.maximum(y * scale[0] + sh[0], 0.0)
    f = jnp.max(h, axis=1)
    B, N, _ = f.shape
    t = f.reshape(B * N, C)
    u_ref[...] = jnp.dot(t, wu_ref[...],
                         preferred_element_type=jnp.float32).reshape(
                             B, N, -1)
    vh_ref[...] = jnp.dot(t, wvh_ref[...],
                          preferred_element_type=jnp.float32).reshape(
                              B, N, -1)
    vs_ref[...] = (jnp.dot(t, wvs_ref[...],
                           preferred_element_type=jnp.float32)
                   + sb_ref[...]).reshape(B, N, -1)


def _ep1_uv2(e, v, g, bt, wu, wvh, wvs, sb):
    B, N, C = v.shape
    return pl.pallas_call(
        _ep1_body,
        out_shape=[
            jax.ShapeDtypeStruct((B, N, wu.shape[1]), jnp.float32),
            jax.ShapeDtypeStruct((B, N, wvh.shape[1]), jnp.float32),
            jax.ShapeDtypeStruct((B, N, wvs.shape[1]), jnp.float32),
        ],
    )(e, v, g, bt, wu, wvh, wvs, sb)


def _alpha_of(ee, vs_ref, w_ref, Ch):
    ys = ee[0, :, :, Ch:] + vs_ref[0][None]
    l = jnp.sum(jnp.maximum(ys, 0.0) * w_ref[...][0], axis=-1) * (1.0 / _TAU)
    mx = jnp.max(l, axis=0, keepdims=True)
    ex = jnp.exp(l - mx)
    return ex / jnp.sum(ex, axis=0, keepdims=True)


def _stats_accum(ee, vh_ref, sm_ref, first, Ch):
    yh = ee[0, :, :, 0:Ch] + vh_ref[0][None]
    y2 = yh.reshape(-1, Ch)
    ones = jnp.ones((1, y2.shape[0]), jnp.float32)
    s1 = lax.dot_general(ones, y2, (((1,), (0,)), ((), ())),
                         preferred_element_type=jnp.float32)
    s2 = lax.dot_general(ones, y2 * y2, (((1,), (0,)), ((), ())),
                         preferred_element_type=jnp.float32)

    @pl.when(first)
    def _():
        sm_ref[...] = jnp.zeros_like(sm_ref)

    sm_ref[0:1, :] += s1
    sm_ref[1:2, :] += s2


def _combine(ee, vh_ref, vs_ref, w_ref, sm_ref, g_ref, bt_ref, minv, Ch):
    al = _alpha_of(ee, vs_ref, w_ref, Ch)
    m = sm_ref[0:1, :] * minv
    var = sm_ref[1:2, :] * minv - m * m
    scale = g_ref[...] * lax.rsqrt(var + _EPS)
    sh = bt_ref[...] - m * scale
    yh = ee[0, :, :, 0:Ch] + vh_ref[0][None]
    h = jnp.maximum(yh * scale[0] + sh[0], 0.0)
    return jnp.sum(h * al[..., None], axis=0)


def _st2_body(e_ref, vh_ref, vs_ref, w_ref, g_ref, bt_ref,
              wu_ref, wvh_ref, wvs_ref, sb_ref,
              sm_ref, u_ref, uvh_ref, uvs_ref, *, minv, Ch):
    s = pl.program_id(0)

    @pl.when(s == 0)
    def _():
        ee = e_ref[...]
        first = jnp.logical_and(pl.program_id(1) == 0, pl.program_id(2) == 0)
        _stats_accum(ee, vh_ref, sm_ref, first, Ch)

    @pl.when(s == 1)
    def _():
        ee = e_ref[...]
        f = _combine(ee, vh_ref, vs_ref, w_ref, sm_ref, g_ref, bt_ref,
                     minv, Ch)
        u_ref[0] = jnp.dot(f, wu_ref[...],
                           preferred_element_type=jnp.float32)
        uvh_ref[0] = jnp.dot(f, wvh_ref[...],
                             preferred_element_type=jnp.float32)
        uvs_ref[0] = (jnp.dot(f, wvs_ref[...],
                              preferred_element_type=jnp.float32)
                      + sb_ref[...])


def _stage2(e, vh, vs, s2w, g, bt, wu, wvh, wvs, sb, nb, Ch):
    B, _, N, Cp = e.shape
    nch = N // nb
    minv = 1.0 / (B * _K * N)
    Cu, Cvh, Cvs = wu.shape[1], wvh.shape[1], wvs.shape[1]
    specs = [
        pl.BlockSpec((1, _K, nb, Cp), lambda s, b, i: (b, 0, i, 0)),
        pl.BlockSpec((1, nb, Ch), lambda s, b, i: (b, i, 0)),
        pl.BlockSpec((1, nb, 128), lambda s, b, i: (b, i, 0)),
        pl.BlockSpec((1, 128), lambda s, b, i: (0, 0)),
        pl.BlockSpec((1, Ch), lambda s, b, i: (0, 0)),
        pl.BlockSpec((1, Ch), lambda s, b, i: (0, 0)),
        pl.BlockSpec((Ch, Cu), lambda s, b, i: (0, 0)),
        pl.BlockSpec((Ch, Cvh), lambda s, b, i: (0, 0)),
        pl.BlockSpec((Ch, Cvs), lambda s, b, i: (0, 0)),
        pl.BlockSpec((1, Cvs), lambda s, b, i: (0, 0)),
    ]
    return pl.pallas_call(
        functools.partial(_st2_body, minv=minv, Ch=Ch),
        grid=(2, B, nch),
        in_specs=specs,
        out_specs=[
            pl.BlockSpec((8, Ch), lambda s, b, i: (0, 0)),
            pl.BlockSpec((1, nb, Cu), lambda s, b, i: (b, i, 0)),
            pl.BlockSpec((1, nb, Cvh), lambda s, b, i: (b, i, 0)),
            pl.BlockSpec((1, nb, Cvs), lambda s, b, i: (b, i, 0)),
        ],
        out_shape=[
            jax.ShapeDtypeStruct((8, Ch), jnp.float32),
            jax.ShapeDtypeStruct((B, N, Cu), jnp.float32),
            jax.ShapeDtypeStruct((B, N, Cvh), jnp.float32),
            jax.ShapeDtypeStruct((B, N, Cvs), jnp.float32),
        ],
    )(e, vh, vs, s2w, g, bt, wu, wvh, wvs, sb)[1:]


def _st3_body(e_ref, vh_ref, vs_ref, w_ref, g_ref, bt_ref,
              w1_ref, b1_ref, w2_ref, fcw_ref, fcb_ref,
              sm_ref, o_ref, *, minv, Ch):
    s = pl.program_id(0)

    @pl.when(s == 0)
    def _():
        ee = e_ref[...]
        _stats_accum(ee, vh_ref, sm_ref, pl.program_id(1) == 0, Ch)

    @pl.when(s == 1)
    def _():
        ee = e_ref[...]
        f = _combine(ee, vh_ref, vs_ref, w_ref, sm_ref, g_ref, bt_ref,
                     minv, Ch)
        a1 = jnp.maximum(
            jnp.dot(f, w1_ref[...], preferred_element_type=jnp.float32)
            + b1_ref[...], 0.0)
        sc = jnp.sum(a1 * w2_ref[...], axis=1, keepdims=True)
        mx = jnp.max(sc, axis=0, keepdims=True)
        ex = jnp.exp(sc - mx)
        w = ex / jnp.sum(ex, axis=0, keepdims=True)
        pooled = lax.dot_general(w, f, (((0,), (0,)), ((), ())),
                                 preferred_element_type=jnp.float32)
        out = jnp.maximum(
            jnp.dot(pooled, fcw_ref[...], preferred_element_type=jnp.float32)
            + fcb_ref[...], 0.0)
        o_ref[0] = jnp.broadcast_to(out, (8, out.shape[-1]))


def _stage3_head(e, vh, vs, s2w, g, bt, w1t, b1, w2, fcwt, fcb):
    B, _, N, Cp = e.shape
    Ch = vh.shape[-1]
    A = w1t.shape[1]
    minv = 1.0 / (B * _K * N)
    specs = [
        pl.BlockSpec((1, _K, N, Cp), lambda s, b: (b, 0, 0, 0)),
        pl.BlockSpec((1, N, Ch), lambda s, b: (b, 0, 0)),
        pl.BlockSpec((1, N, 128), lambda s, b: (b, 0, 0)),
        pl.BlockSpec((1, 128), lambda s, b: (0, 0)),
        pl.BlockSpec((1, Ch), lambda s, b: (0, 0)),
        pl.BlockSpec((1, Ch), lambda s, b: (0, 0)),
        pl.BlockSpec((Ch, A), lambda s, b: (0, 0)),
        pl.BlockSpec((1, A), lambda s, b: (0, 0)),
        pl.BlockSpec((1, A), lambda s, b: (0, 0)),
        pl.BlockSpec((Ch, Ch), lambda s, b: (0, 0)),
        pl.BlockSpec((1, Ch), lambda s, b: (0, 0)),
    ]
    return pl.pallas_call(
        functools.partial(_st3_body, minv=minv, Ch=Ch),
        grid=(2, B),
        in_specs=specs,
        out_specs=[
            pl.BlockSpec((8, Ch), lambda s, b: (0, 0)),
            pl.BlockSpec((1, 8, Ch), lambda s, b: (b, 0, 0)),
        ],
        out_shape=[
            jax.ShapeDtypeStruct((8, Ch), jnp.float32),
            jax.ShapeDtypeStruct((B, 8, Ch), jnp.float32),
        ],
    )(e, vh, vs, s2w, g, bt, w1t, b1, w2, fcwt, fcb)[1][:, 0, :]


def kernel(x, ec1_w, ec1_g, ec1_b, ec2_mw, ec2_g, ec2_b, ec2_s1w, ec2_s1b,
           ec2_s2w, ec2_s2b, ec3_mw, ec3_g, ec3_b, ec3_s1w, ec3_s1b, ec3_s2w,
           ec3_s2b, att_w1, att_b1, att_w2, att_b2, fc_w, fc_b):
    B, N, _ = x.shape
    M = B * _K * N

    wg1 = jnp.concatenate([ec1_w[:, 0:3], ec1_w[:, 6:8]], axis=1)
    wc1 = jnp.concatenate([ec1_w[:, 3:6], ec1_w[:, 8:10]], axis=1)
    wg1t = jnp.pad(wg1.T, ((0, 0), (0, 64)))
    idx, U1, V1 = _knn_uv1(x, wg1t, (wc1 - wg1).T)
    idx3 = idx.reshape(B, 16 * N)

    def split(mw, s1w, s1b, s2w, Cin):
        wgh, wch = mw[:, :Cin], mw[:, Cin:] - mw[:, :Cin]
        wgs, wcs = s1w[:, :Cin], s1w[:, Cin:] - s1w[:, :Cin]
        Cs = wgs.shape[0]
        if Cs % 128:
            p = 128 - Cs % 128
            wgs = jnp.pad(wgs, ((0, p), (0, 0)))
            wcs = jnp.pad(wcs, ((0, p), (0, 0)))
            s1b = jnp.pad(s1b, (0, p))
            s2w = jnp.pad(s2w, ((0, 0), (0, p)))
        wu = jnp.concatenate([wgh.T, wgs.T], axis=1)
        return wu, wch.T, wcs.T, s1b[None], s2w

    wu2, wvh2, wvs2, sb2, s2w2 = split(ec2_mw, ec2_s1w, ec2_s1b, ec2_s2w, 64)
    wu3, wvh3, wvs3, sb3, s2w3 = split(ec3_mw, ec3_s1w, ec3_s1b, ec3_s2w, 128)

    E1 = _sc_gather(idx3, U1.reshape(B * N, -1), 0)
    U2, V2h, V2s = _ep1_uv2(E1.reshape(B, _K, N, -1), V1,
                            ec1_g[None], ec1_b[None], wu2, wvh2, wvs2, sb2)

    E2 = _sc_gather(idx3, U2.reshape(B * N, -1), 1).reshape(B, _K, N, -1)
    U3, V3h, V3s = _stage2(E2, V2h, V2s, s2w2, ec2_g[None], ec2_b[None],
                           wu3, wvh3, wvs3, sb3, 512, 128)

    E3 = _sc_gather(idx3, U3.reshape(B * N, -1), 1).reshape(B, _K, N, -1)
    return _stage3_head(E3, V3h, V3s, s2w3, ec3_g[None], ec3_b[None],
                        att_w1.T, att_b1[None], att_w2, fc_w.T, fc_b[None])


# speedup vs baseline: 5.2741x; 1.0537x over previous
import functools

import jax
import jax.numpy as jnp
from jax import lax
from jax.experimental import pallas as pl
from jax.experimental.pallas import tpu as pltpu
from jax.experimental.pallas import tpu_sc as plsc

_K = 10
_TAU = 0.2
_EPS = 1e-5


def _knn_uv1_body(x_ref, wg_ref, wc_ref, idx_ref, u_ref, v_ref):
    b = pl.program_id(0)
    N = x_ref.shape[1]
    t = x_ref[0]
    lane = lax.broadcasted_iota(jnp.int32, t.shape, 1)
    c = jnp.where(lane < 3, t, 0.0)
    sq = jnp.sum(c * c, axis=1, keepdims=True)
    dot = lax.dot_general(c, c, (((1,), (1,)), ((), ())),
                          preferred_element_type=jnp.float32)
    row = lax.broadcasted_iota(jnp.int32, (N, N), 0)
    col = lax.broadcasted_iota(jnp.int32, (N, N), 1)
    v = jnp.where(row == col, jnp.float32(1e30), sq - 2.0 * dot)
    base = b * N
    idx_ref[0, 0:1, :] = lax.broadcasted_iota(jnp.int32, (1, N), 1) + base
    for j in range(1, _K + 1):
        mn = jnp.min(v, axis=0, keepdims=True)
        am = jnp.min(jnp.where(v <= mn, row, N), axis=0, keepdims=True)
        idx_ref[0, j:j + 1, :] = am + base
        v = jnp.where(row == am, jnp.float32(1e30), v)
    idx_ref[0, _K + 1:16, :] = jnp.full((16 - _K - 1, N), base, jnp.int32)
    u_ref[0] = jnp.dot(t, wg_ref[...], preferred_element_type=jnp.float32)
    v_ref[0] = jnp.dot(t, wc_ref[...], preferred_element_type=jnp.float32)


def _knn_uv1(x, wg1t, wc1t):
    B, N, _ = x.shape
    Cu, Cv = wg1t.shape[1], wc1t.shape[1]
    return pl.pallas_call(
        _knn_uv1_body,
        grid=(B,),
        in_specs=[
            pl.BlockSpec((1, N, 5), lambda b: (b, 0, 0)),
            pl.BlockSpec((5, Cu), lambda b: (0, 0)),
            pl.BlockSpec((5, Cv), lambda b: (0, 0)),
        ],
        out_specs=[
            pl.BlockSpec((1, 16, N), lambda b: (b, 0, 0)),
            pl.BlockSpec((1, N, Cu), lambda b: (b, 0, 0)),
            pl.BlockSpec((1, N, Cv), lambda b: (b, 0, 0)),
        ],
        out_shape=[
            jax.ShapeDtypeStruct((B, 16, N), jnp.int32),
            jax.ShapeDtypeStruct((B, N, Cu), jnp.float32),
            jax.ShapeDtypeStruct((B, N, Cv), jnp.float32),
        ],
    )(x, wg1t, wc1t)


_CH = 128
_NW = 32


def _sc_gather(idx3, tb, j0):
    B = idx3.shape[0]
    C = tb.shape[1]
    N = tb.shape[0] // B
    M = B * _K * N
    per_w = (M // _CH) // _NW
    wpb = _NW // B
    mesh = plsc.VectorSubcoreMesh(core_axis_name="c", subcore_axis_name="s")

    @functools.partial(
        pl.kernel,
        mesh=mesh,
        out_type=jax.ShapeDtypeStruct((M, C), jnp.float32),
        scratch_types=[
            pltpu.VMEM((per_w * _CH,), jnp.int32),
            pltpu.VMEM((_CH, C), jnp.float32),
            pltpu.VMEM((_CH, C), jnp.float32),
            pltpu.SemaphoreType.DMA,
            pltpu.SemaphoreType.DMA,
            pltpu.SemaphoreType.DMA,
        ],
    )
    def k(idx_hbm, tb_hbm, o_hbm, idx_v, buf_a, buf_b, sg, sw0, sw1):
        wid = lax.axis_index("s") * 2 + lax.axis_index("c")
        b = wid // wpb
        woff = wid % wpb
        off0 = j0 * N + woff * per_w * _CH
        out0 = b * _K * N + woff * per_w * _CH
        pltpu.sync_copy(idx_hbm.at[b, pl.ds(off0, per_w * _CH)], idx_v)
        bufs = (buf_a, buf_b)
        sws = (sw0, sw1)

        def gstart(r):
            return pltpu.async_copy(
                tb_hbm.at[idx_v.at[pl.ds(r * _CH, _CH)]], bufs[r % 2], sg)

        def wstart(r):
            return pltpu.async_copy(
                bufs[r % 2], o_hbm.at[pl.ds(out0 + r * _CH, _CH), :],
                sws[r % 2])

        g_prev = gstart(0)
        w = [None] * per_w
        for r in range(1, per_w):
            g_prev.wait()
            w[r - 1] = wstart(r - 1)
            if r >= 2:
                w[r - 2].wait()
            g_prev = gstart(r)
        g_prev.wait()
        w[per_w - 1] = wstart(per_w - 1)
        w[per_w - 2].wait()
        w[per_w - 1].wait()

    return k(idx3, tb)


def _sc_gather2(idx3, th, ts, j0):
    B = idx3.shape[0]
    Ch, Cs = th.shape[1], ts.shape[1]
    N = th.shape[0] // B
    M = B * _K * N
    per_w = (M // _CH) // _NW
    wpb = _NW // B
    mesh = plsc.VectorSubcoreMesh(core_axis_name="c", subcore_axis_name="s")

    @functools.partial(
        pl.kernel,
        mesh=mesh,
        out_type=(jax.ShapeDtypeStruct((M, Ch), jnp.float32),
                  jax.ShapeDtypeStruct((M, Cs), jnp.float32)),
        scratch_types=[
            pltpu.VMEM((per_w * _CH,), jnp.int32),
            pltpu.VMEM((_CH, Ch), jnp.float32),
            pltpu.VMEM((_CH, Ch), jnp.float32),
            pltpu.VMEM((_CH, Cs), jnp.float32),
            pltpu.VMEM((_CH, Cs), jnp.float32),
            pltpu.SemaphoreType.DMA,
            pltpu.SemaphoreType.DMA,
            pltpu.SemaphoreType.DMA,
            pltpu.SemaphoreType.DMA,
            pltpu.SemaphoreType.DMA,
            pltpu.SemaphoreType.DMA,
        ],
    )
    def k(idx_hbm, th_hbm, ts_hbm, oh_hbm, os_hbm, idx_v,
          bha, bhb, bsa, bsb, sgh, sgs, swh0, swh1, sws0, sws1):
        wid = lax.axis_index("s") * 2 + lax.axis_index("c")
        b = wid // wpb
        woff = wid % wpb
        off0 = j0 * N + woff * per_w * _CH
        out0 = b * _K * N + woff * per_w * _CH
        pltpu.sync_copy(idx_hbm.at[b, pl.ds(off0, per_w * _CH)], idx_v)
        bh = (bha, bhb)
        bs = (bsa, bsb)
        swh = (swh0, swh1)
        sws = (sws0, sws1)

        def gstart(r):
            ix = idx_v.at[pl.ds(r * _CH, _CH)]
            return (pltpu.async_copy(th_hbm.at[ix], bh[r % 2], sgh),
                    pltpu.async_copy(ts_hbm.at[ix], bs[r % 2], sgs))

        def wstart(r):
            sl = pl.ds(out0 + r * _CH, _CH)
            return (pltpu.async_copy(bh[r % 2], oh_hbm.at[sl, :], swh[r % 2]),
                    pltpu.async_copy(bs[r % 2], os_hbm.at[sl, :], sws[r % 2]))

        def waitall(pair):
            pair[0].wait()
            pair[1].wait()

        g_prev = gstart(0)
        w = [None] * per_w
        for r in range(1, per_w):
            waitall(g_prev)
            w[r - 1] = wstart(r - 1)
            if r >= 2:
                waitall(w[r - 2])
            g_prev = gstart(r)
        waitall(g_prev)
        w[per_w - 1] = wstart(per_w - 1)
        waitall(w[per_w - 2])
        waitall(w[per_w - 1])

    return k(idx3, th, ts)


def _ep1_body(e_ref, v_ref, g_ref, bt_ref, wuh_ref, wus_ref, wvh_ref,
              wvs_ref, sb_ref, uh_ref, us_ref, vh_ref, vs_ref):
    C = v_ref.shape[-1]
    y = e_ref[...][..., 0:C] + v_ref[...][:, None]
    y2 = y.reshape(-1, C)
    minv = 1.0 / y2.shape[0]
    ones = jnp.ones((1, y2.shape[0]), jnp.float32)
    m = lax.dot_general(ones, y2, (((1,), (0,)), ((), ())),
                        preferred_element_type=jnp.float32) * minv
    var = lax.dot_general(ones, y2 * y2, (((1,), (0,)), ((), ())),
                          preferred_element_type=jnp.float32) * minv - m * m
    scale = g_ref[...] * lax.rsqrt(var + _EPS)
    sh = bt_ref[...] - m * scale
    h = jnp.maximum(y * scale[0] + sh[0], 0.0)
    f = jnp.max(h, axis=1)
    B, N, _ = f.shape
    t = f.reshape(B * N, C)
    uh_ref[...] = jnp.dot(t, wuh_ref[...],
                          preferred_element_type=jnp.float32).reshape(
                              B, N, -1)
    us_ref[...] = jnp.dot(t, wus_ref[...],
                          preferred_element_type=jnp.float32).reshape(
                              B, N, -1)
    vh_ref[...] = jnp.dot(t, wvh_ref[...],
                          preferred_element_type=jnp.float32).reshape(
                              B, N, -1)
    vs_ref[...] = (jnp.dot(t, wvs_ref[...],
                           preferred_element_type=jnp.float32)
                   + sb_ref[...]).reshape(B, N, -1)


def _ep1_uv2(e, v, g, bt, wuh, wus, wvh, wvs, sb):
    B, N, C = v.shape
    return pl.pallas_call(
        _ep1_body,
        out_shape=[
            jax.ShapeDtypeStruct((B, N, wuh.shape[1]), jnp.float32),
            jax.ShapeDtypeStruct((B, N, wus.shape[1]), jnp.float32),
            jax.ShapeDtypeStruct((B, N, wvh.shape[1]), jnp.float32),
            jax.ShapeDtypeStruct((B, N, wvs.shape[1]), jnp.float32),
        ],
    )(e, v, g, bt, wuh, wus, wvh, wvs, sb)


def _alpha_of(es, vs_ref, w_ref):
    ys = es[0] + vs_ref[0][None]
    l = jnp.sum(jnp.maximum(ys, 0.0) * w_ref[...][0], axis=-1) * (1.0 / _TAU)
    mx = jnp.max(l, axis=0, keepdims=True)
    ex = jnp.exp(l - mx)
    return ex / jnp.sum(ex, axis=0, keepdims=True)


def _stats_accum(eh, vh_ref, sm_ref, first, Ch):
    yh = eh[0] + vh_ref[0][None]
    y2 = yh.reshape(-1, Ch)
    ones = jnp.ones((1, y2.shape[0]), jnp.float32)
    s1 = lax.dot_general(ones, y2, (((1,), (0,)), ((), ())),
                         preferred_element_type=jnp.float32)
    s2 = lax.dot_general(ones, y2 * y2, (((1,), (0,)), ((), ())),
                         preferred_element_type=jnp.float32)

    @pl.when(first)
    def _():
        sm_ref[...] = jnp.zeros_like(sm_ref)

    sm_ref[0:1, :] += s1
    sm_ref[1:2, :] += s2


def _combine(eh, es, vh_ref, vs_ref, w_ref, sm_ref, g_ref, bt_ref, minv, Ch):
    al = _alpha_of(es, vs_ref, w_ref)
    m = sm_ref[0:1, :] * minv
    var = sm_ref[1:2, :] * minv - m * m
    scale = g_ref[...] * lax.rsqrt(var + _EPS)
    sh = bt_ref[...] - m * scale
    yh = eh[0] + vh_ref[0][None]
    h = jnp.maximum(yh * scale[0] + sh[0], 0.0)
    return jnp.sum(h * al[..., None], axis=0)


def _st2_body(eh_ref, es_ref, vh_ref, vs_ref, w_ref, g_ref, bt_ref,
              wuh_ref, wus_ref, wvh_ref, wvs_ref, sb_ref,
              sm_ref, uh_ref, us_ref, uvh_ref, uvs_ref, *, minv, Ch):
    s = pl.program_id(0)

    @pl.when(s == 0)
    def _():
        first = jnp.logical_and(pl.program_id(1) == 0, pl.program_id(2) == 0)
        _stats_accum(eh_ref[...], vh_ref, sm_ref, first, Ch)

    @pl.when(s == 1)
    def _():
        f = _combine(eh_ref[...], es_ref[...], vh_ref, vs_ref, w_ref,
                     sm_ref, g_ref, bt_ref, minv, Ch)
        uh_ref[0] = jnp.dot(f, wuh_ref[...],
                            preferred_element_type=jnp.float32)
        us_ref[0] = jnp.dot(f, wus_ref[...],
                            preferred_element_type=jnp.float32)
        uvh_ref[0] = jnp.dot(f, wvh_ref[...],
                             preferred_element_type=jnp.float32)
        uvs_ref[0] = (jnp.dot(f, wvs_ref[...],
                              preferred_element_type=jnp.float32)
                      + sb_ref[...])


def _stage2(eh, es, vh, vs, s2w, g, bt, wuh, wus, wvh, wvs, sb, nb, Ch):
    B, _, N, _ = eh.shape
    nch = N // nb
    minv = 1.0 / (B * _K * N)
    Cuh, Cus = wuh.shape[1], wus.shape[1]
    Cvh, Cvs = wvh.shape[1], wvs.shape[1]
    specs = [
        pl.BlockSpec((1, _K, nb, Ch), lambda s, b, i: (b, 0, i, 0)),
        pl.BlockSpec((1, _K, nb, 128), lambda s, b, i: (b * s, 0, i * s, 0)),
        pl.BlockSpec((1, nb, Ch), lambda s, b, i: (b, i, 0)),
        pl.BlockSpec((1, nb, 128), lambda s, b, i: (b * s, i * s, 0)),
        pl.BlockSpec((1, 128), lambda s, b, i: (0, 0)),
        pl.BlockSpec((1, Ch), lambda s, b, i: (0, 0)),
        pl.BlockSpec((1, Ch), lambda s, b, i: (0, 0)),
        pl.BlockSpec((Ch, Cuh), lambda s, b, i: (0, 0)),
        pl.BlockSpec((Ch, Cus), lambda s, b, i: (0, 0)),
        pl.BlockSpec((Ch, Cvh), lambda s, b, i: (0, 0)),
        pl.BlockSpec((Ch, Cvs), lambda s, b, i: (0, 0)),
        pl.BlockSpec((1, Cvs), lambda s, b, i: (0, 0)),
    ]
    return pl.pallas_call(
        functools.partial(_st2_body, minv=minv, Ch=Ch),
        grid=(2, B, nch),
        in_specs=specs,
        out_specs=[
            pl.BlockSpec((8, Ch), lambda s, b, i: (0, 0)),
            pl.BlockSpec((1, nb, Cuh), lambda s, b, i: (b, i, 0)),
            pl.BlockSpec((1, nb, Cus), lambda s, b, i: (b, i, 0)),
            pl.BlockSpec((1, nb, Cvh), lambda s, b, i: (b, i, 0)),
            pl.BlockSpec((1, nb, Cvs), lambda s, b, i: (b, i, 0)),
        ],
        out_shape=[
            jax.ShapeDtypeStruct((8, Ch), jnp.float32),
            jax.ShapeDtypeStruct((B, N, Cuh), jnp.float32),
            jax.ShapeDtypeStruct((B, N, Cus), jnp.float32),
            jax.ShapeDtypeStruct((B, N, Cvh), jnp.float32),
            jax.ShapeDtypeStruct((B, N, Cvs), jnp.float32),
        ],
    )(eh, es, vh, vs, s2w, g, bt, wuh, wus, wvh, wvs, sb)[1:]


def _st3_body(eh_ref, es_ref, vh_ref, vs_ref, w_ref, g_ref, bt_ref,
              w1_ref, b1_ref, w2_ref, fcw_ref, fcb_ref,
              sm_ref, o_ref, *, minv, Ch):
    s = pl.program_id(0)

    @pl.when(s == 0)
    def _():
        _stats_accum(eh_ref[...], vh_ref, sm_ref, pl.program_id(1) == 0, Ch)

    @pl.when(s == 1)
    def _():
        f = _combine(eh_ref[...], es_ref[...], vh_ref, vs_ref, w_ref,
                     sm_ref, g_ref, bt_ref, minv, Ch)
        a1 = jnp.maximum(
            jnp.dot(f, w1_ref[...], preferred_element_type=jnp.float32)
            + b1_ref[...], 0.0)
        sc = jnp.sum(a1 * w2_ref[...], axis=1, keepdims=True)
        mx = jnp.max(sc, axis=0, keepdims=True)
        ex = jnp.exp(sc - mx)
        w = ex / jnp.sum(ex, axis=0, keepdims=True)
        pooled = lax.dot_general(w, f, (((0,), (0,)), ((), ())),
                                 preferred_element_type=jnp.float32)
        out = jnp.maximum(
            jnp.dot(pooled, fcw_ref[...], preferred_element_type=jnp.float32)
            + fcb_ref[...], 0.0)
        o_ref[0] = jnp.broadcast_to(out, (8, out.shape[-1]))


def _stage3_head(eh, es, vh, vs, s2w, g, bt, w1t, b1, w2, fcwt, fcb):
    B, _, N, _ = eh.shape
    Ch = vh.shape[-1]
    A = w1t.shape[1]
    minv = 1.0 / (B * _K * N)
    specs = [
        pl.BlockSpec((1, _K, N, Ch), lambda s, b: (b, 0, 0, 0)),
        pl.BlockSpec((1, _K, N, 128), lambda s, b: (b * s, 0, 0, 0)),
        pl.BlockSpec((1, N, Ch), lambda s, b: (b, 0, 0)),
        pl.BlockSpec((1, N, 128), lambda s, b: (b * s, 0, 0)),
        pl.BlockSpec((1, 128), lambda s, b: (0, 0)),
        pl.BlockSpec((1, Ch), lambda s, b: (0, 0)),
        pl.BlockSpec((1, Ch), lambda s, b: (0, 0)),
        pl.BlockSpec((Ch, A), lambda s, b: (0, 0)),
        pl.BlockSpec((1, A), lambda s, b: (0, 0)),
        pl.BlockSpec((1, A), lambda s, b: (0, 0)),
        pl.BlockSpec((Ch, Ch), lambda s, b: (0, 0)),
        pl.BlockSpec((1, Ch), lambda s, b: (0, 0)),
    ]
    return pl.pallas_call(
        functools.partial(_st3_body, minv=minv, Ch=Ch),
        grid=(2, B),
        in_specs=specs,
        out_specs=[
            pl.BlockSpec((8, Ch), lambda s, b: (0, 0)),
            pl.BlockSpec((1, 8, Ch), lambda s, b: (b, 0, 0)),
        ],
        out_shape=[
            jax.ShapeDtypeStruct((8, Ch), jnp.float32),
            jax.ShapeDtypeStruct((B, 8, Ch), jnp.float32),
        ],
    )(eh, es, vh, vs, s2w, g, bt, w1t, b1, w2, fcwt, fcb)[1][:, 0, :]


def kernel(x, ec1_w, ec1_g, ec1_b, ec2_mw, ec2_g, ec2_b, ec2_s1w, ec2_s1b,
           ec2_s2w, ec2_s2b, ec3_mw, ec3_g, ec3_b, ec3_s1w, ec3_s1b, ec3_s2w,
           ec3_s2b, att_w1, att_b1, att_w2, att_b2, fc_w, fc_b):
    B, N, _ = x.shape
    M = B * _K * N

    wg1 = jnp.concatenate([ec1_w[:, 0:3], ec1_w[:, 6:8]], axis=1)
    wc1 = jnp.concatenate([ec1_w[:, 3:6], ec1_w[:, 8:10]], axis=1)
    wg1t = jnp.pad(wg1.T, ((0, 0), (0, 64)))
    idx, U1, V1 = _knn_uv1(x, wg1t, (wc1 - wg1).T)
    idx3 = idx.reshape(B, 16 * N)

    def split(mw, s1w, s1b, s2w, Cin):
        wgh, wch = mw[:, :Cin], mw[:, Cin:] - mw[:, :Cin]
        wgs, wcs = s1w[:, :Cin], s1w[:, Cin:] - s1w[:, :Cin]
        Cs = wgs.shape[0]
        if Cs % 128:
            p = 128 - Cs % 128
            wgs = jnp.pad(wgs, ((0, p), (0, 0)))
            wcs = jnp.pad(wcs, ((0, p), (0, 0)))
            s1b = jnp.pad(s1b, (0, p))
            s2w = jnp.pad(s2w, ((0, 0), (0, p)))
        return wgh.T, wgs.T, wch.T, wcs.T, s1b[None], s2w

    wuh2, wus2, wvh2, wvs2, sb2, s2w2 = split(ec2_mw, ec2_s1w, ec2_s1b,
                                              ec2_s2w, 64)
    wuh3, wus3, wvh3, wvs3, sb3, s2w3 = split(ec3_mw, ec3_s1w, ec3_s1b,
                                              ec3_s2w, 128)

    E1 = _sc_gather(idx3, U1.reshape(B * N, -1), 0)
    U2h, U2s, V2h, V2s = _ep1_uv2(E1.reshape(B, _K, N, -1), V1,
                                  ec1_g[None], ec1_b[None],
                                  wuh2, wus2, wvh2, wvs2, sb2)

    E2h, E2s = _sc_gather2(idx3, U2h.reshape(B * N, -1),
                           U2s.reshape(B * N, -1), 1)
    E2h = E2h.reshape(B, _K, N, -1)
    E2s = E2s.reshape(B, _K, N, -1)
    U3h, U3s, V3h, V3s = _stage2(E2h, E2s, V2h, V2s, s2w2,
                                 ec2_g[None], ec2_b[None],
                                 wuh3, wus3, wvh3, wvs3, sb3, 512, 128)

    E3h, E3s = _sc_gather2(idx3, U3h.reshape(B * N, -1),
                           U3s.reshape(B * N, -1), 1)
    E3h = E3h.reshape(B, _K, N, -1)
    E3s = E3s.reshape(B, _K, N, -1)
    return _stage3_head(E3h, E3s, V3h, V3s, s2w3, ec3_g[None], ec3_b[None],
                        att_w1.T, att_b1[None], att_w2, fc_w.T, fc_b[None])


# speedup vs baseline: 5.3046x; 1.0058x over previous
import functools

import jax
import jax.numpy as jnp
from jax import lax
from jax.experimental import pallas as pl
from jax.experimental.pallas import tpu as pltpu
from jax.experimental.pallas import tpu_sc as plsc

_K = 10
_TAU = 0.2
_EPS = 1e-5


def _knn_uv1_body(x_ref, wg_ref, wc_ref, idx_ref, u_ref, v_ref):
    b = pl.program_id(0)
    N = x_ref.shape[1]
    t = x_ref[0]
    lane = lax.broadcasted_iota(jnp.int32, t.shape, 1)
    c = jnp.where(lane < 3, t, 0.0)
    sq = jnp.sum(c * c, axis=1, keepdims=True)
    dot = lax.dot_general(c, c, (((1,), (1,)), ((), ())),
                          preferred_element_type=jnp.float32)
    row = lax.broadcasted_iota(jnp.int32, (N, N), 0)
    col = lax.broadcasted_iota(jnp.int32, (N, N), 1)
    v = jnp.where(row == col, jnp.float32(1e30), sq - 2.0 * dot)
    base = b * N
    idx_ref[0, 0:1, :] = lax.broadcasted_iota(jnp.int32, (1, N), 1) + base
    for j in range(1, _K + 1):
        mn = jnp.min(v, axis=0, keepdims=True)
        am = jnp.min(jnp.where(v <= mn, row, N), axis=0, keepdims=True)
        idx_ref[0, j:j + 1, :] = am + base
        v = jnp.where(row == am, jnp.float32(1e30), v)
    idx_ref[0, _K + 1:16, :] = jnp.full((16 - _K - 1, N), base, jnp.int32)
    u_ref[0] = jnp.dot(t, wg_ref[...], preferred_element_type=jnp.float32)
    v_ref[0] = jnp.dot(t, wc_ref[...], preferred_element_type=jnp.float32)


def _knn_uv1(x, wg1t, wc1t):
    B, N, _ = x.shape
    Cu, Cv = wg1t.shape[1], wc1t.shape[1]
    return pl.pallas_call(
        _knn_uv1_body,
        grid=(B,),
        in_specs=[
            pl.BlockSpec((1, N, 5), lambda b: (b, 0, 0)),
            pl.BlockSpec((5, Cu), lambda b: (0, 0)),
            pl.BlockSpec((5, Cv), lambda b: (0, 0)),
        ],
        out_specs=[
            pl.BlockSpec((1, 16, N), lambda b: (b, 0, 0)),
            pl.BlockSpec((1, N, Cu), lambda b: (b, 0, 0)),
            pl.BlockSpec((1, N, Cv), lambda b: (b, 0, 0)),
        ],
        out_shape=[
            jax.ShapeDtypeStruct((B, 16, N), jnp.int32),
            jax.ShapeDtypeStruct((B, N, Cu), jnp.float32),
            jax.ShapeDtypeStruct((B, N, Cv), jnp.float32),
        ],
    )(x, wg1t, wc1t)


_CH = 128
_NW = 32


def _sc_gather(idx3, tb, j0):
    B = idx3.shape[0]
    C = tb.shape[1]
    N = tb.shape[0] // B
    M = B * _K * N
    per_w = (M // _CH) // _NW
    wpb = _NW // B
    mesh = plsc.VectorSubcoreMesh(core_axis_name="c", subcore_axis_name="s")

    @functools.partial(
        pl.kernel,
        mesh=mesh,
        out_type=jax.ShapeDtypeStruct((M, C), jnp.float32),
        scratch_types=[
            pltpu.VMEM((per_w * _CH,), jnp.int32),
            pltpu.VMEM((_CH, C), jnp.float32),
            pltpu.VMEM((_CH, C), jnp.float32),
            pltpu.SemaphoreType.DMA,
            pltpu.SemaphoreType.DMA,
            pltpu.SemaphoreType.DMA,
        ],
    )
    def k(idx_hbm, tb_hbm, o_hbm, idx_v, buf_a, buf_b, sg, sw0, sw1):
        wid = lax.axis_index("s") * 2 + lax.axis_index("c")
        b = wid // wpb
        woff = wid % wpb
        off0 = j0 * N + woff * per_w * _CH
        out0 = b * _K * N + woff * per_w * _CH
        pltpu.sync_copy(idx_hbm.at[b, pl.ds(off0, per_w * _CH)], idx_v)
        bufs = (buf_a, buf_b)
        sws = (sw0, sw1)

        def gstart(r):
            return pltpu.async_copy(
                tb_hbm.at[idx_v.at[pl.ds(r * _CH, _CH)]], bufs[r % 2], sg)

        def wstart(r):
            return pltpu.async_copy(
                bufs[r % 2], o_hbm.at[pl.ds(out0 + r * _CH, _CH), :],
                sws[r % 2])

        g_prev = gstart(0)
        w = [None] * per_w
        for r in range(1, per_w):
            g_prev.wait()
            w[r - 1] = wstart(r - 1)
            if r >= 2:
                w[r - 2].wait()
            g_prev = gstart(r)
        g_prev.wait()
        w[per_w - 1] = wstart(per_w - 1)
        w[per_w - 2].wait()
        w[per_w - 1].wait()

    return k(idx3, tb)


def _sc_gather2(idx3, th, ts, j0):
    B = idx3.shape[0]
    Ch, Cs = th.shape[1], ts.shape[1]
    N = th.shape[0] // B
    M = B * _K * N
    per_w = (M // _CH) // _NW
    wpb = _NW // B
    mesh = plsc.VectorSubcoreMesh(core_axis_name="c", subcore_axis_name="s")

    @functools.partial(
        pl.kernel,
        mesh=mesh,
        out_type=(jax.ShapeDtypeStruct((M, Ch), jnp.float32),
                  jax.ShapeDtypeStruct((M, Cs), jnp.float32)),
        scratch_types=[
            pltpu.VMEM((per_w * _CH,), jnp.int32),
            pltpu.VMEM((_CH, Ch), jnp.float32),
            pltpu.VMEM((_CH, Ch), jnp.float32),
            pltpu.VMEM((_CH, Cs), jnp.float32),
            pltpu.VMEM((_CH, Cs), jnp.float32),
            pltpu.SemaphoreType.DMA,
            pltpu.SemaphoreType.DMA,
            pltpu.SemaphoreType.DMA,
            pltpu.SemaphoreType.DMA,
            pltpu.SemaphoreType.DMA,
            pltpu.SemaphoreType.DMA,
        ],
    )
    def k(idx_hbm, th_hbm, ts_hbm, oh_hbm, os_hbm, idx_v,
          bha, bhb, bsa, bsb, sgh, sgs, swh0, swh1, sws0, sws1):
        wid = lax.axis_index("s") * 2 + lax.axis_index("c")
        b = wid // wpb
        woff = wid % wpb
        off0 = j0 * N + woff * per_w * _CH
        out0 = b * _K * N + woff * per_w * _CH
        pltpu.sync_copy(idx_hbm.at[b, pl.ds(off0, per_w * _CH)], idx_v)
        bh = (bha, bhb)
        bs = (bsa, bsb)
        swh = (swh0, swh1)
        sws = (sws0, sws1)

        def gstart(r):
            ix = idx_v.at[pl.ds(r * _CH, _CH)]
            return (pltpu.async_copy(th_hbm.at[ix], bh[r % 2], sgh),
                    pltpu.async_copy(ts_hbm.at[ix], bs[r % 2], sgs))

        def wstart(r):
            sl = pl.ds(out0 + r * _CH, _CH)
            return (pltpu.async_copy(bh[r % 2], oh_hbm.at[sl, :], swh[r % 2]),
                    pltpu.async_copy(bs[r % 2], os_hbm.at[sl, :], sws[r % 2]))

        def waitall(pair):
            pair[0].wait()
            pair[1].wait()

        g_prev = gstart(0)
        w = [None] * per_w
        for r in range(1, per_w):
            waitall(g_prev)
            w[r - 1] = wstart(r - 1)
            if r >= 2:
                waitall(w[r - 2])
            g_prev = gstart(r)
        waitall(g_prev)
        w[per_w - 1] = wstart(per_w - 1)
        waitall(w[per_w - 2])
        waitall(w[per_w - 1])

    return k(idx3, th, ts)


def _ep1_body(e_ref, v_ref, g_ref, bt_ref, wuh_ref, wus_ref, wvh_ref,
              wvs_ref, sb_ref, uh_ref, us_ref, vh_ref, vs_ref):
    C = v_ref.shape[-1]
    y = e_ref[...][..., 0:C] + v_ref[...][:, None]
    y2 = y.reshape(-1, C)
    minv = 1.0 / y2.shape[0]
    ones = jnp.ones((1, y2.shape[0]), jnp.float32)
    m = lax.dot_general(ones, y2, (((1,), (0,)), ((), ())),
                        preferred_element_type=jnp.float32) * minv
    var = lax.dot_general(ones, y2 * y2, (((1,), (0,)), ((), ())),
                          preferred_element_type=jnp.float32) * minv - m * m
    scale = g_ref[...] * lax.rsqrt(var + _EPS)
    sh = bt_ref[...] - m * scale
    h = jnp.maximum(y * scale[0] + sh[0], 0.0)
    f = jnp.max(h, axis=1)
    B, N, _ = f.shape
    t = f.reshape(B * N, C)
    uh_ref[...] = jnp.dot(t, wuh_ref[...],
                          preferred_element_type=jnp.float32).reshape(
                              B, N, -1)
    us_ref[...] = jnp.dot(t, wus_ref[...],
                          preferred_element_type=jnp.float32).reshape(
                              B, N, -1)
    vh_ref[...] = jnp.dot(t, wvh_ref[...],
                          preferred_element_type=jnp.float32).reshape(
                              B, N, -1)
    vs_ref[...] = (jnp.dot(t, wvs_ref[...],
                           preferred_element_type=jnp.float32)
                   + sb_ref[...]).reshape(B, N, -1)


def _ep1_uv2(e, v, g, bt, wuh, wus, wvh, wvs, sb):
    B, N, C = v.shape
    return pl.pallas_call(
        _ep1_body,
        out_shape=[
            jax.ShapeDtypeStruct((B, N, wuh.shape[1]), jnp.float32),
            jax.ShapeDtypeStruct((B, N, wus.shape[1]), jnp.float32),
            jax.ShapeDtypeStruct((B, N, wvh.shape[1]), jnp.float32),
            jax.ShapeDtypeStruct((B, N, wvs.shape[1]), jnp.float32),
        ],
    )(e, v, g, bt, wuh, wus, wvh, wvs, sb)


def _alpha_of(es, vs_ref, w_ref):
    ys = es[0] + vs_ref[0][None]
    l = jnp.sum(jnp.maximum(ys, 0.0) * w_ref[...][0], axis=-1) * (1.0 / _TAU)
    mx = jnp.max(l, axis=0, keepdims=True)
    ex = jnp.exp(l - mx)
    return ex / jnp.sum(ex, axis=0, keepdims=True)


def _stats_accum(eh, vh_ref, sm_ref, first, Ch):
    yh = eh[0] + vh_ref[0][None]
    y2 = yh.reshape(-1, Ch)
    ones = jnp.ones((1, y2.shape[0]), jnp.float32)
    s1 = lax.dot_general(ones, y2, (((1,), (0,)), ((), ())),
                         preferred_element_type=jnp.float32)
    s2 = lax.dot_general(ones, y2 * y2, (((1,), (0,)), ((), ())),
                         preferred_element_type=jnp.float32)

    @pl.when(first)
    def _():
        sm_ref[...] = jnp.zeros_like(sm_ref)

    sm_ref[0:1, :] += s1
    sm_ref[1:2, :] += s2


def _combine(eh, es, vh_ref, vs_ref, w_ref, sm_ref, g_ref, bt_ref, minv, Ch):
    al = _alpha_of(es, vs_ref, w_ref)
    m = sm_ref[0:1, :] * minv
    var = sm_ref[1:2, :] * minv - m * m
    scale = g_ref[...] * lax.rsqrt(var + _EPS)
    sh = bt_ref[...] - m * scale
    yh = eh[0] + vh_ref[0][None]
    h = jnp.maximum(yh * scale[0] + sh[0], 0.0)
    return jnp.sum(h * al[..., None], axis=0)


def _st2_body(eh_ref, es_ref, vh_ref, vs_ref, w_ref, g_ref, bt_ref,
              wuh_ref, wus_ref, wvh_ref, wvs_ref, sb_ref,
              sm_ref, uh_ref, us_ref, uvh_ref, uvs_ref, *, minv, Ch):
    s = pl.program_id(0)

    @pl.when(s == 0)
    def _():
        first = jnp.logical_and(pl.program_id(1) == 0, pl.program_id(2) == 0)
        _stats_accum(eh_ref[...], vh_ref, sm_ref, first, Ch)

    @pl.when(s == 1)
    def _():
        f = _combine(eh_ref[...], es_ref[...], vh_ref, vs_ref, w_ref,
                     sm_ref, g_ref, bt_ref, minv, Ch)
        uh_ref[0] = jnp.dot(f, wuh_ref[...],
                            preferred_element_type=jnp.float32)
        us_ref[0] = jnp.dot(f, wus_ref[...],
                            preferred_element_type=jnp.float32)
        uvh_ref[0] = jnp.dot(f, wvh_ref[...],
                             preferred_element_type=jnp.float32)
        uvs_ref[0] = (jnp.dot(f, wvs_ref[...],
                              preferred_element_type=jnp.float32)
                      + sb_ref[...])


def _stage2(eh, es, vh, vs, s2w, g, bt, wuh, wus, wvh, wvs, sb, nb, Ch):
    B, _, N, _ = eh.shape
    nch = N // nb
    minv = 1.0 / (B * _K * N)
    Cuh, Cus = wuh.shape[1], wus.shape[1]
    Cvh, Cvs = wvh.shape[1], wvs.shape[1]
    specs = [
        pl.BlockSpec((1, _K, nb, Ch), lambda s, b, i: (b, 0, i, 0)),
        pl.BlockSpec((1, _K, nb, 128), lambda s, b, i: (b * s, 0, i * s, 0)),
        pl.BlockSpec((1, nb, Ch), lambda s, b, i: (b, i, 0)),
        pl.BlockSpec((1, nb, 128), lambda s, b, i: (b * s, i * s, 0)),
        pl.BlockSpec((1, 128), lambda s, b, i: (0, 0)),
        pl.BlockSpec((1, Ch), lambda s, b, i: (0, 0)),
        pl.BlockSpec((1, Ch), lambda s, b, i: (0, 0)),
        pl.BlockSpec((Ch, Cuh), lambda s, b, i: (0, 0)),
        pl.BlockSpec((Ch, Cus), lambda s, b, i: (0, 0)),
        pl.BlockSpec((Ch, Cvh), lambda s, b, i: (0, 0)),
        pl.BlockSpec((Ch, Cvs), lambda s, b, i: (0, 0)),
        pl.BlockSpec((1, Cvs), lambda s, b, i: (0, 0)),
    ]
    return pl.pallas_call(
        functools.partial(_st2_body, minv=minv, Ch=Ch),
        grid=(2, B, nch),
        in_specs=specs,
        out_specs=[
            pl.BlockSpec((8, Ch), lambda s, b, i: (0, 0)),
            pl.BlockSpec((1, nb, Cuh), lambda s, b, i: (b, i, 0)),
            pl.BlockSpec((1, nb, Cus), lambda s, b, i: (b, i, 0)),
            pl.BlockSpec((1, nb, Cvh), lambda s, b, i: (b, i, 0)),
            pl.BlockSpec((1, nb, Cvs), lambda s, b, i: (b, i, 0)),
        ],
        out_shape=[
            jax.ShapeDtypeStruct((8, Ch), jnp.float32),
            jax.ShapeDtypeStruct((B, N, Cuh), jnp.float32),
            jax.ShapeDtypeStruct((B, N, Cus), jnp.float32),
            jax.ShapeDtypeStruct((B, N, Cvh), jnp.float32),
            jax.ShapeDtypeStruct((B, N, Cvs), jnp.float32),
        ],
    )(eh, es, vh, vs, s2w, g, bt, wuh, wus, wvh, wvs, sb)[1:]


def _st3_body(eh_ref, es_ref, vh_ref, vs_ref, w_ref, g_ref, bt_ref,
              w1_ref, b1_ref, w2_ref, fcw_ref, fcb_ref,
              sm_ref, o_ref, *, minv, Ch):
    s = pl.program_id(0)

    @pl.when(s == 0)
    def _():
        _stats_accum(eh_ref[...], vh_ref, sm_ref, pl.program_id(1) == 0, Ch)

    @pl.when(s == 1)
    def _():
        f = _combine(eh_ref[...], es_ref[...], vh_ref, vs_ref, w_ref,
                     sm_ref, g_ref, bt_ref, minv, Ch)
        a1 = jnp.maximum(
            jnp.dot(f, w1_ref[...], preferred_element_type=jnp.float32)
            + b1_ref[...], 0.0)
        sc = jnp.sum(a1 * w2_ref[...], axis=1, keepdims=True)
        mx = jnp.max(sc, axis=0, keepdims=True)
        ex = jnp.exp(sc - mx)
        w = ex / jnp.sum(ex, axis=0, keepdims=True)
        pooled = lax.dot_general(w, f, (((0,), (0,)), ((), ())),
                                 preferred_element_type=jnp.float32)
        out = jnp.maximum(
            jnp.dot(pooled, fcw_ref[...], preferred_element_type=jnp.float32)
            + fcb_ref[...], 0.0)
        o_ref[0] = jnp.broadcast_to(out, (8, out.shape[-1]))


def _stage3_head(eh, es, vh, vs, s2w, g, bt, w1t, b1, w2, fcwt, fcb):
    B, _, N, _ = eh.shape
    Ch = vh.shape[-1]
    A = w1t.shape[1]
    minv = 1.0 / (B * _K * N)
    specs = [
        pl.BlockSpec((1, _K, N, Ch), lambda s, b: (b, 0, 0, 0)),
        pl.BlockSpec((1, _K, N, 128), lambda s, b: (b * s, 0, 0, 0)),
        pl.BlockSpec((1, N, Ch), lambda s, b: (b, 0, 0)),
        pl.BlockSpec((1, N, 128), lambda s, b: (b * s, 0, 0)),
        pl.BlockSpec((1, 128), lambda s, b: (0, 0)),
        pl.BlockSpec((1, Ch), lambda s, b: (0, 0)),
        pl.BlockSpec((1, Ch), lambda s, b: (0, 0)),
        pl.BlockSpec((Ch, A), lambda s, b: (0, 0)),
        pl.BlockSpec((1, A), lambda s, b: (0, 0)),
        pl.BlockSpec((1, A), lambda s, b: (0, 0)),
        pl.BlockSpec((Ch, Ch), lambda s, b: (0, 0)),
        pl.BlockSpec((1, Ch), lambda s, b: (0, 0)),
    ]
    return pl.pallas_call(
        functools.partial(_st3_body, minv=minv, Ch=Ch),
        grid=(2, B),
        in_specs=specs,
        out_specs=[
            pl.BlockSpec((8, Ch), lambda s, b: (0, 0)),
            pl.BlockSpec((1, 8, Ch), lambda s, b: (b, 0, 0)),
        ],
        out_shape=[
            jax.ShapeDtypeStruct((8, Ch), jnp.float32),
            jax.ShapeDtypeStruct((B, 8, Ch), jnp.float32),
        ],
    )(eh, es, vh, vs, s2w, g, bt, w1t, b1, w2, fcwt, fcb)[1][:, 0, :]


def kernel(x, ec1_w, ec1_g, ec1_b, ec2_mw, ec2_g, ec2_b, ec2_s1w, ec2_s1b,
           ec2_s2w, ec2_s2b, ec3_mw, ec3_g, ec3_b, ec3_s1w, ec3_s1b, ec3_s2w,
           ec3_s2b, att_w1, att_b1, att_w2, att_b2, fc_w, fc_b):
    B, N, _ = x.shape
    M = B * _K * N

    wg1 = jnp.concatenate([ec1_w[:, 0:3], ec1_w[:, 6:8]], axis=1)
    wc1 = jnp.concatenate([ec1_w[:, 3:6], ec1_w[:, 8:10]], axis=1)
    wg1t = jnp.pad(wg1.T, ((0, 0), (0, 64)))
    idx, U1, V1 = _knn_uv1(x, wg1t, (wc1 - wg1).T)
    idx3 = idx.reshape(B, 16 * N)

    def split(mw, s1w, s1b, s2w, Cin):
        wgh, wch = mw[:, :Cin], mw[:, Cin:] - mw[:, :Cin]
        wgs, wcs = s1w[:, :Cin], s1w[:, Cin:] - s1w[:, :Cin]
        Cs = wgs.shape[0]
        if Cs % 128:
            p = 128 - Cs % 128
            wgs = jnp.pad(wgs, ((0, p), (0, 0)))
            wcs = jnp.pad(wcs, ((0, p), (0, 0)))
            s1b = jnp.pad(s1b, (0, p))
            s2w = jnp.pad(s2w, ((0, 0), (0, p)))
        return wgh.T, wgs.T, wch.T, wcs.T, s1b[None], s2w

    wuh2, wus2, wvh2, wvs2, sb2, s2w2 = split(ec2_mw, ec2_s1w, ec2_s1b,
                                              ec2_s2w, 64)
    wuh3, wus3, wvh3, wvs3, sb3, s2w3 = split(ec3_mw, ec3_s1w, ec3_s1b,
                                              ec3_s2w, 128)

    E1 = _sc_gather(idx3, U1.reshape(B * N, -1), 0)
    U2h, U2s, V2h, V2s = _ep1_uv2(E1.reshape(B, _K, N, -1), V1,
                                  ec1_g[None], ec1_b[None],
                                  wuh2, wus2, wvh2, wvs2, sb2)

    E2h, E2s = _sc_gather2(idx3, U2h.reshape(B * N, -1),
                           U2s.reshape(B * N, -1), 1)
    E2h = E2h.reshape(B, _K, N, -1)
    E2s = E2s.reshape(B, _K, N, -1)
    U3h, U3s, V3h, V3s = _stage2(E2h, E2s, V2h, V2s, s2w2,
                                 ec2_g[None], ec2_b[None],
                                 wuh3, wus3, wvh3, wvs3, sb3, 1024, 128)

    E3h, E3s = _sc_gather2(idx3, U3h.reshape(B * N, -1),
                           U3s.reshape(B * N, -1), 1)
    E3h = E3h.reshape(B, _K, N, -1)
    E3s = E3s.reshape(B, _K, N, -1)
    return _stage3_head(E3h, E3s, V3h, V3s, s2w3, ec3_g[None], ec3_b[None],
                        att_w1.T, att_b1[None], att_w2, fc_w.T, fc_b[None])


# speedup vs baseline: 5.4026x; 1.0185x over previous
import functools

import jax
import jax.numpy as jnp
from jax import lax
from jax.experimental import pallas as pl
from jax.experimental.pallas import tpu as pltpu
from jax.experimental.pallas import tpu_sc as plsc

_K = 10
_TAU = 0.2
_EPS = 1e-5


def _knn_uv1_body(x_ref, wg_ref, wc_ref, idx_ref, u_ref, v_ref):
    b = pl.program_id(0)
    N = x_ref.shape[1]
    t = x_ref[0]
    lane = lax.broadcasted_iota(jnp.int32, t.shape, 1)
    c = jnp.where(lane < 3, t, 0.0)
    sq = jnp.sum(c * c, axis=1, keepdims=True)
    dot = lax.dot_general(c, c, (((1,), (1,)), ((), ())),
                          preferred_element_type=jnp.float32)
    row = lax.broadcasted_iota(jnp.int32, (N, N), 0)
    col = lax.broadcasted_iota(jnp.int32, (N, N), 1)
    v = jnp.where(row == col, jnp.float32(1e30), sq - 2.0 * dot)
    base = b * N
    idx_ref[0, 0:1, :] = lax.broadcasted_iota(jnp.int32, (1, N), 1) + base
    for j in range(1, _K + 1):
        mn = jnp.min(v, axis=0, keepdims=True)
        am = jnp.min(jnp.where(v <= mn, row, N), axis=0, keepdims=True)
        idx_ref[0, j:j + 1, :] = am + base
        v = jnp.where(row == am, jnp.float32(1e30), v)
    idx_ref[0, _K + 1:16, :] = jnp.full((16 - _K - 1, N), base, jnp.int32)
    u_ref[0] = jnp.dot(t, wg_ref[...], preferred_element_type=jnp.float32)
    v_ref[0] = jnp.dot(t, wc_ref[...], preferred_element_type=jnp.float32)


def _knn_uv1(x, wg1t, wc1t):
    B, N, _ = x.shape
    Cu, Cv = wg1t.shape[1], wc1t.shape[1]
    return pl.pallas_call(
        _knn_uv1_body,
        grid=(B,),
        in_specs=[
            pl.BlockSpec((1, N, 5), lambda b: (b, 0, 0)),
            pl.BlockSpec((5, Cu), lambda b: (0, 0)),
            pl.BlockSpec((5, Cv), lambda b: (0, 0)),
        ],
        out_specs=[
            pl.BlockSpec((1, 16, N), lambda b: (b, 0, 0)),
            pl.BlockSpec((1, N, Cu), lambda b: (b, 0, 0)),
            pl.BlockSpec((1, N, Cv), lambda b: (b, 0, 0)),
        ],
        out_shape=[
            jax.ShapeDtypeStruct((B, 16, N), jnp.int32),
            jax.ShapeDtypeStruct((B, N, Cu), jnp.float32),
            jax.ShapeDtypeStruct((B, N, Cv), jnp.float32),
        ],
    )(x, wg1t, wc1t)


_CH = 128
_NW = 32


def _sc_gather(idx3, tb, j0):
    B = idx3.shape[0]
    C = tb.shape[1]
    N = tb.shape[0] // B
    M = B * _K * N
    per_w = (M // _CH) // _NW
    wpb = _NW // B
    mesh = plsc.VectorSubcoreMesh(core_axis_name="c", subcore_axis_name="s")

    @functools.partial(
        pl.kernel,
        mesh=mesh,
        out_type=jax.ShapeDtypeStruct((M, C), jnp.float32),
        scratch_types=[
            pltpu.VMEM((per_w * _CH,), jnp.int32),
            pltpu.VMEM((_CH, C), jnp.float32),
            pltpu.VMEM((_CH, C), jnp.float32),
            pltpu.SemaphoreType.DMA,
            pltpu.SemaphoreType.DMA,
            pltpu.SemaphoreType.DMA,
            pltpu.SemaphoreType.DMA,
        ],
    )
    def k(idx_hbm, tb_hbm, o_hbm, idx_v, buf_a, buf_b, sg0, sg1, sw0, sw1):
        wid = lax.axis_index("s") * 2 + lax.axis_index("c")
        b = wid // wpb
        woff = wid % wpb
        off0 = j0 * N + woff * per_w * _CH
        out0 = b * _K * N + woff * per_w * _CH
        pltpu.sync_copy(idx_hbm.at[b, pl.ds(off0, per_w * _CH)], idx_v)
        bufs = (buf_a, buf_b)
        sgs = (sg0, sg1)
        sws = (sw0, sw1)

        def gstart(r):
            return pltpu.async_copy(
                tb_hbm.at[idx_v.at[pl.ds(r * _CH, _CH)]], bufs[r % 2],
                sgs[r % 2])

        def wstart(r):
            return pltpu.async_copy(
                bufs[r % 2], o_hbm.at[pl.ds(out0 + r * _CH, _CH), :],
                sws[r % 2])

        g = [None] * per_w
        w = [None] * per_w
        g[0] = gstart(0)
        for r in range(1, per_w):
            if r >= 2:
                w[r - 2].wait()
            g[r] = gstart(r)
            g[r - 1].wait()
            w[r - 1] = wstart(r - 1)
        g[per_w - 1].wait()
        w[per_w - 2].wait()
        w[per_w - 1] = wstart(per_w - 1)
        w[per_w - 1].wait()

    return k(idx3, tb)


def _sc_gather2(idx3, th, ts, j0):
    B = idx3.shape[0]
    Ch, Cs = th.shape[1], ts.shape[1]
    N = th.shape[0] // B
    M = B * _K * N
    per_w = (M // _CH) // _NW
    wpb = _NW // B
    mesh = plsc.VectorSubcoreMesh(core_axis_name="c", subcore_axis_name="s")

    @functools.partial(
        pl.kernel,
        mesh=mesh,
        out_type=(jax.ShapeDtypeStruct((M, Ch), jnp.float32),
                  jax.ShapeDtypeStruct((M, Cs), jnp.float32)),
        scratch_types=[
            pltpu.VMEM((per_w * _CH,), jnp.int32),
            pltpu.VMEM((_CH, Ch), jnp.float32),
            pltpu.VMEM((_CH, Ch), jnp.float32),
            pltpu.VMEM((_CH, Cs), jnp.float32),
            pltpu.VMEM((_CH, Cs), jnp.float32),
            pltpu.SemaphoreType.DMA,
            pltpu.SemaphoreType.DMA,
            pltpu.SemaphoreType.DMA,
            pltpu.SemaphoreType.DMA,
            pltpu.SemaphoreType.DMA,
            pltpu.SemaphoreType.DMA,
            pltpu.SemaphoreType.DMA,
            pltpu.SemaphoreType.DMA,
        ],
    )
    def k(idx_hbm, th_hbm, ts_hbm, oh_hbm, os_hbm, idx_v,
          bha, bhb, bsa, bsb, sgh0, sgh1, sgs0, sgs1, swh0, swh1, sws0, sws1):
        wid = lax.axis_index("s") * 2 + lax.axis_index("c")
        b = wid // wpb
        woff = wid % wpb
        off0 = j0 * N + woff * per_w * _CH
        out0 = b * _K * N + woff * per_w * _CH
        pltpu.sync_copy(idx_hbm.at[b, pl.ds(off0, per_w * _CH)], idx_v)
        bh = (bha, bhb)
        bs = (bsa, bsb)
        sgh = (sgh0, sgh1)
        sgs = (sgs0, sgs1)
        swh = (swh0, swh1)
        sws = (sws0, sws1)

        def gstart(r):
            ix = idx_v.at[pl.ds(r * _CH, _CH)]
            return (pltpu.async_copy(th_hbm.at[ix], bh[r % 2], sgh[r % 2]),
                    pltpu.async_copy(ts_hbm.at[ix], bs[r % 2], sgs[r % 2]))

        def wstart(r):
            sl = pl.ds(out0 + r * _CH, _CH)
            return (pltpu.async_copy(bh[r % 2], oh_hbm.at[sl, :], swh[r % 2]),
                    pltpu.async_copy(bs[r % 2], os_hbm.at[sl, :], sws[r % 2]))

        def waitall(pair):
            pair[0].wait()
            pair[1].wait()

        g = [None] * per_w
        w = [None] * per_w
        g[0] = gstart(0)
        for r in range(1, per_w):
            if r >= 2:
                waitall(w[r - 2])
            g[r] = gstart(r)
            waitall(g[r - 1])
            w[r - 1] = wstart(r - 1)
        waitall(g[per_w - 1])
        waitall(w[per_w - 2])
        w[per_w - 1] = wstart(per_w - 1)
        waitall(w[per_w - 1])

    return k(idx3, th, ts)


def _ep1_body(e_ref, v_ref, g_ref, bt_ref, wuh_ref, wus_ref, wvh_ref,
              wvs_ref, sb_ref, uh_ref, us_ref, vh_ref, vs_ref):
    C = v_ref.shape[-1]
    y = e_ref[...][..., 0:C] + v_ref[...][:, None]
    y2 = y.reshape(-1, C)
    minv = 1.0 / y2.shape[0]
    ones = jnp.ones((1, y2.shape[0]), jnp.float32)
    m = lax.dot_general(ones, y2, (((1,), (0,)), ((), ())),
                        preferred_element_type=jnp.float32) * minv
    var = lax.dot_general(ones, y2 * y2, (((1,), (0,)), ((), ())),
                          preferred_element_type=jnp.float32) * minv - m * m
    scale = g_ref[...] * lax.rsqrt(var + _EPS)
    sh = bt_ref[...] - m * scale
    h = jnp.maximum(y * scale[0] + sh[0], 0.0)
    f = jnp.max(h, axis=1)
    B, N, _ = f.shape
    t = f.reshape(B * N, C)
    uh_ref[...] = jnp.dot(t, wuh_ref[...],
                          preferred_element_type=jnp.float32).reshape(
                              B, N, -1)
    us_ref[...] = jnp.dot(t, wus_ref[...],
                          preferred_element_type=jnp.float32).reshape(
                              B, N, -1)
    vh_ref[...] = jnp.dot(t, wvh_ref[...],
                          preferred_element_type=jnp.float32).reshape(
                              B, N, -1)
    vs_ref[...] = (jnp.dot(t, wvs_ref[...],
                           preferred_element_type=jnp.float32)
                   + sb_ref[...]).reshape(B, N, -1)


def _ep1_uv2(e, v, g, bt, wuh, wus, wvh, wvs, sb):
    B, N, C = v.shape
    return pl.pallas_call(
        _ep1_body,
        out_shape=[
            jax.ShapeDtypeStruct((B, N, wuh.shape[1]), jnp.float32),
            jax.ShapeDtypeStruct((B, N, wus.shape[1]), jnp.float32),
            jax.ShapeDtypeStruct((B, N, wvh.shape[1]), jnp.float32),
            jax.ShapeDtypeStruct((B, N, wvs.shape[1]), jnp.float32),
        ],
    )(e, v, g, bt, wuh, wus, wvh, wvs, sb)


def _alpha_of(es, vs_ref, w_ref):
    ys = es[0] + vs_ref[0][None]
    l = jnp.sum(jnp.maximum(ys, 0.0) * w_ref[...][0], axis=-1) * (1.0 / _TAU)
    mx = jnp.max(l, axis=0, keepdims=True)
    ex = jnp.exp(l - mx)
    return ex / jnp.sum(ex, axis=0, keepdims=True)


def _stats_accum(eh, vh_ref, sm_ref, first, Ch):
    yh = eh[0] + vh_ref[0][None]
    y2 = yh.reshape(-1, Ch)
    ones = jnp.ones((1, y2.shape[0]), jnp.float32)
    s1 = lax.dot_general(ones, y2, (((1,), (0,)), ((), ())),
                         preferred_element_type=jnp.float32)
    s2 = lax.dot_general(ones, y2 * y2, (((1,), (0,)), ((), ())),
                         preferred_element_type=jnp.float32)

    @pl.when(first)
    def _():
        sm_ref[...] = jnp.zeros_like(sm_ref)

    sm_ref[0:1, :] += s1
    sm_ref[1:2, :] += s2


def _combine(eh, es, vh_ref, vs_ref, w_ref, sm_ref, g_ref, bt_ref, minv, Ch):
    al = _alpha_of(es, vs_ref, w_ref)
    m = sm_ref[0:1, :] * minv
    var = sm_ref[1:2, :] * minv - m * m
    scale = g_ref[...] * lax.rsqrt(var + _EPS)
    sh = bt_ref[...] - m * scale
    yh = eh[0] + vh_ref[0][None]
    h = jnp.maximum(yh * scale[0] + sh[0], 0.0)
    return jnp.sum(h * al[..., None], axis=0)


def _st2_body(eh_ref, es_ref, vh_ref, vs_ref, w_ref, g_ref, bt_ref,
              wuh_ref, wus_ref, wvh_ref, wvs_ref, sb_ref,
              sm_ref, uh_ref, us_ref, uvh_ref, uvs_ref, *, minv, Ch):
    s = pl.program_id(0)

    @pl.when(s == 0)
    def _():
        first = jnp.logical_and(pl.program_id(1) == 0, pl.program_id(2) == 0)
        _stats_accum(eh_ref[...], vh_ref, sm_ref, first, Ch)

    @pl.when(s == 1)
    def _():
        f = _combine(eh_ref[...], es_ref[...], vh_ref, vs_ref, w_ref,
                     sm_ref, g_ref, bt_ref, minv, Ch)
        uh_ref[0] = jnp.dot(f, wuh_ref[...],
                            preferred_element_type=jnp.float32)
        us_ref[0] = jnp.dot(f, wus_ref[...],
                            preferred_element_type=jnp.float32)
        uvh_ref[0] = jnp.dot(f, wvh_ref[...],
                             preferred_element_type=jnp.float32)
        uvs_ref[0] = (jnp.dot(f, wvs_ref[...],
                              preferred_element_type=jnp.float32)
                      + sb_ref[...])


def _stage2(eh, es, vh, vs, s2w, g, bt, wuh, wus, wvh, wvs, sb, nb, Ch):
    B, _, N, _ = eh.shape
    nch = N // nb
    minv = 1.0 / (B * _K * N)
    Cuh, Cus = wuh.shape[1], wus.shape[1]
    Cvh, Cvs = wvh.shape[1], wvs.shape[1]
    specs = [
        pl.BlockSpec((1, _K, nb, Ch), lambda s, b, i: (b, 0, i, 0)),
        pl.BlockSpec((1, _K, nb, 128), lambda s, b, i: (b * s, 0, i * s, 0)),
        pl.BlockSpec((1, nb, Ch), lambda s, b, i: (b, i, 0)),
        pl.BlockSpec((1, nb, 128), lambda s, b, i: (b * s, i * s, 0)),
        pl.BlockSpec((1, 128), lambda s, b, i: (0, 0)),
        pl.BlockSpec((1, Ch), lambda s, b, i: (0, 0)),
        pl.BlockSpec((1, Ch), lambda s, b, i: (0, 0)),
        pl.BlockSpec((Ch, Cuh), lambda s, b, i: (0, 0)),
        pl.BlockSpec((Ch, Cus), lambda s, b, i: (0, 0)),
        pl.BlockSpec((Ch, Cvh), lambda s, b, i: (0, 0)),
        pl.BlockSpec((Ch, Cvs), lambda s, b, i: (0, 0)),
        pl.BlockSpec((1, Cvs), lambda s, b, i: (0, 0)),
    ]
    return pl.pallas_call(
        functools.partial(_st2_body, minv=minv, Ch=Ch),
        grid=(2, B, nch),
        in_specs=specs,
        out_specs=[
            pl.BlockSpec((8, Ch), lambda s, b, i: (0, 0)),
            pl.BlockSpec((1, nb, Cuh), lambda s, b, i: (b, i, 0)),
            pl.BlockSpec((1, nb, Cus), lambda s, b, i: (b, i, 0)),
            pl.BlockSpec((1, nb, Cvh), lambda s, b, i: (b, i, 0)),
            pl.BlockSpec((1, nb, Cvs), lambda s, b, i: (b, i, 0)),
        ],
        out_shape=[
            jax.ShapeDtypeStruct((8, Ch), jnp.float32),
            jax.ShapeDtypeStruct((B, N, Cuh), jnp.float32),
            jax.ShapeDtypeStruct((B, N, Cus), jnp.float32),
            jax.ShapeDtypeStruct((B, N, Cvh), jnp.float32),
            jax.ShapeDtypeStruct((B, N, Cvs), jnp.float32),
        ],
    )(eh, es, vh, vs, s2w, g, bt, wuh, wus, wvh, wvs, sb)[1:]


def _st3_body(eh_ref, es_ref, vh_ref, vs_ref, w_ref, g_ref, bt_ref,
              w1_ref, b1_ref, w2_ref, fcw_ref, fcb_ref,
              sm_ref, o_ref, *, minv, Ch):
    s = pl.program_id(0)

    @pl.when(s == 0)
    def _():
        _stats_accum(eh_ref[...], vh_ref, sm_ref, pl.program_id(1) == 0, Ch)

    @pl.when(s == 1)
    def _():
        f = _combine(eh_ref[...], es_ref[...], vh_ref, vs_ref, w_ref,
                     sm_ref, g_ref, bt_ref, minv, Ch)
        a1 = jnp.maximum(
            jnp.dot(f, w1_ref[...], preferred_element_type=jnp.float32)
            + b1_ref[...], 0.0)
        sc = jnp.sum(a1 * w2_ref[...], axis=1, keepdims=True)
        mx = jnp.max(sc, axis=0, keepdims=True)
        ex = jnp.exp(sc - mx)
        w = ex / jnp.sum(ex, axis=0, keepdims=True)
        pooled = lax.dot_general(w, f, (((0,), (0,)), ((), ())),
                                 preferred_element_type=jnp.float32)
        out = jnp.maximum(
            jnp.dot(pooled, fcw_ref[...], preferred_element_type=jnp.float32)
            + fcb_ref[...], 0.0)
        o_ref[0] = jnp.broadcast_to(out, (8, out.shape[-1]))


def _stage3_head(eh, es, vh, vs, s2w, g, bt, w1t, b1, w2, fcwt, fcb):
    B, _, N, _ = eh.shape
    Ch = vh.shape[-1]
    A = w1t.shape[1]
    minv = 1.0 / (B * _K * N)
    specs = [
        pl.BlockSpec((1, _K, N, Ch), lambda s, b: (b, 0, 0, 0)),
        pl.BlockSpec((1, _K, N, 128), lambda s, b: (b * s, 0, 0, 0)),
        pl.BlockSpec((1, N, Ch), lambda s, b: (b, 0, 0)),
        pl.BlockSpec((1, N, 128), lambda s, b: (b * s, 0, 0)),
        pl.BlockSpec((1, 128), lambda s, b: (0, 0)),
        pl.BlockSpec((1, Ch), lambda s, b: (0, 0)),
        pl.BlockSpec((1, Ch), lambda s, b: (0, 0)),
        pl.BlockSpec((Ch, A), lambda s, b: (0, 0)),
        pl.BlockSpec((1, A), lambda s, b: (0, 0)),
        pl.BlockSpec((1, A), lambda s, b: (0, 0)),
        pl.BlockSpec((Ch, Ch), lambda s, b: (0, 0)),
        pl.BlockSpec((1, Ch), lambda s, b: (0, 0)),
    ]
    return pl.pallas_call(
        functools.partial(_st3_body, minv=minv, Ch=Ch),
        grid=(2, B),
        in_specs=specs,
        out_specs=[
            pl.BlockSpec((8, Ch), lambda s, b: (0, 0)),
            pl.BlockSpec((1, 8, Ch), lambda s, b: (b, 0, 0)),
        ],
        out_shape=[
            jax.ShapeDtypeStruct((8, Ch), jnp.float32),
            jax.ShapeDtypeStruct((B, 8, Ch), jnp.float32),
        ],
    )(eh, es, vh, vs, s2w, g, bt, w1t, b1, w2, fcwt, fcb)[1][:, 0, :]


def kernel(x, ec1_w, ec1_g, ec1_b, ec2_mw, ec2_g, ec2_b, ec2_s1w, ec2_s1b,
           ec2_s2w, ec2_s2b, ec3_mw, ec3_g, ec3_b, ec3_s1w, ec3_s1b, ec3_s2w,
           ec3_s2b, att_w1, att_b1, att_w2, att_b2, fc_w, fc_b):
    B, N, _ = x.shape
    M = B * _K * N

    wg1 = jnp.concatenate([ec1_w[:, 0:3], ec1_w[:, 6:8]], axis=1)
    wc1 = jnp.concatenate([ec1_w[:, 3:6], ec1_w[:, 8:10]], axis=1)
    wg1t = jnp.pad(wg1.T, ((0, 0), (0, 64)))
    idx, U1, V1 = _knn_uv1(x, wg1t, (wc1 - wg1).T)
    idx3 = idx.reshape(B, 16 * N)

    def split(mw, s1w, s1b, s2w, Cin):
        wgh, wch = mw[:, :Cin], mw[:, Cin:] - mw[:, :Cin]
        wgs, wcs = s1w[:, :Cin], s1w[:, Cin:] - s1w[:, :Cin]
        Cs = wgs.shape[0]
        if Cs % 128:
            p = 128 - Cs % 128
            wgs = jnp.pad(wgs, ((0, p), (0, 0)))
            wcs = jnp.pad(wcs, ((0, p), (0, 0)))
            s1b = jnp.pad(s1b, (0, p))
            s2w = jnp.pad(s2w, ((0, 0), (0, p)))
        return wgh.T, wgs.T, wch.T, wcs.T, s1b[None], s2w

    wuh2, wus2, wvh2, wvs2, sb2, s2w2 = split(ec2_mw, ec2_s1w, ec2_s1b,
                                              ec2_s2w, 64)
    wuh3, wus3, wvh3, wvs3, sb3, s2w3 = split(ec3_mw, ec3_s1w, ec3_s1b,
                                              ec3_s2w, 128)

    E1 = _sc_gather(idx3, U1.reshape(B * N, -1), 0)
    U2h, U2s, V2h, V2s = _ep1_uv2(E1.reshape(B, _K, N, -1), V1,
                                  ec1_g[None], ec1_b[None],
                                  wuh2, wus2, wvh2, wvs2, sb2)

    E2h, E2s = _sc_gather2(idx3, U2h.reshape(B * N, -1),
                           U2s.reshape(B * N, -1), 1)
    E2h = E2h.reshape(B, _K, N, -1)
    E2s = E2s.reshape(B, _K, N, -1)
    U3h, U3s, V3h, V3s = _stage2(E2h, E2s, V2h, V2s, s2w2,
                                 ec2_g[None], ec2_b[None],
                                 wuh3, wus3, wvh3, wvs3, sb3, 1024, 128)

    E3h, E3s = _sc_gather2(idx3, U3h.reshape(B * N, -1),
                           U3s.reshape(B * N, -1), 1)
    E3h = E3h.reshape(B, _K, N, -1)
    E3s = E3s.reshape(B, _K, N, -1)
    return _stage3_head(E3h, E3s, V3h, V3s, s2w3, ec3_g[None], ec3_b[None],
                        att_w1.T, att_b1[None], att_w2, fc_w.T, fc_b[None])
